# Initial kernel scaffold; baseline (speedup 1.0000x reference)
import jax
import jax.numpy as jnp
from jax.experimental import pallas as pl


def kernel(data, segment_ids):
    raise NotImplementedError("write your pallas kernel here")


# speedup vs baseline: 68.8474x; 68.8474x over previous
import functools

import jax
import jax.numpy as jnp
from jax import lax
from jax.experimental import pallas as pl
from jax.experimental.pallas import tpu as pltpu
from jax.experimental.pallas import tpu_sc as plsc

N = 1_600_000
C = 50_000

NB = N // 128
BPC = 20
PB = BPC * 128
NCHUNKS = N // PB
NW = 32
TMAX = (NCHUNKS + NW - 1) // NW
RPT = C // 16
ZR = 625

CPAD = 50_176
G = CPAD // 128
GRID = G // 8

_MESH = plsc.VectorSubcoreMesh(core_axis_name="c", subcore_axis_name="s")
_SC_PARAMS = pltpu.CompilerParams(use_tc_tiling_on_sc=False,
                                  needs_layout_passes=False)


def _zero_rows(ref, nrows):
    zero16 = jnp.zeros((16,), jnp.float32)

    def body(i, carry):
        ref[i, :] = zero16
        return carry

    lax.fori_loop(0, nrows, body, 0)


def _sc_prologue(acc, rowbuf, zbuf, sid):
    _zero_rows(rowbuf, 128)
    _zero_rows(zbuf, ZR)
    base = sid * RPT
    for r in range(RPT // ZR):
        pltpu.sync_copy(zbuf, acc.at[pl.ds(base + r * ZR, ZR), :])
    plsc.subcore_barrier()


def _sc_epilogue(acc, out, cid, sid):
    plsc.subcore_barrier()
    base = sid * RPT
    pltpu.sync_copy(acc.at[pl.ds(base, RPT), :],
                    out.at[cid, pl.ds(base, RPT), :])


@functools.partial(
    pl.kernel,
    out_type=jax.ShapeDtypeStruct((2, C, 16), jnp.float32),
    mesh=_MESH,
    scratch_types=[
        pltpu.VMEM_SHARED((C, 16), jnp.float32),
        pltpu.VMEM((PB, 5), jnp.float32),
        pltpu.VMEM((BPC, 128), jnp.int32),
        pltpu.VMEM((128, 16), jnp.float32),
        pltpu.VMEM((ZR, 16), jnp.float32),
    ],
    compiler_params=_SC_PARAMS,
)
def _sc_moments(data_hbm, seg2d_hbm, mom_out, acc, dbuf, sbuf, rowbuf, zbuf):
    cid = lax.axis_index("c")
    sid = lax.axis_index("s")
    wid = cid * 16 + sid
    _sc_prologue(acc, rowbuf, zbuf, sid)
    iota = lax.iota(jnp.int32, 16)
    ones = jnp.full((16,), 1.0, jnp.float32)
    for g in range(8):
        plsc.store_scatter(rowbuf, [iota + g * 16, jnp.zeros((16,), jnp.int32)],
                           ones)

    def chunk_body(t, carry):
        chunk = wid + NW * t

        @pl.when(chunk < NCHUNKS)
        def _():
            pltpu.sync_copy(data_hbm.at[pl.ds(chunk * PB, PB), :], dbuf)
            pltpu.sync_copy(seg2d_hbm.at[pl.ds(chunk * BPC, BPC), :], sbuf)

            def blk_body(b, c2):
                for g in range(8):
                    rows = iota + (b * 128 + g * 16)
                    rr = iota + g * 16
                    vx = plsc.load_gather(dbuf, [rows, jnp.full((16,), 1, jnp.int32)])
                    vy = plsc.load_gather(dbuf, [rows, jnp.full((16,), 2, jnp.int32)])
                    vz = plsc.load_gather(dbuf, [rows, jnp.full((16,), 3, jnp.int32)])

                    def put(col, val):
                        plsc.store_scatter(
                            rowbuf, [rr, jnp.full((16,), col, jnp.int32)], val)

                    put(1, vx)
                    put(2, vy)
                    put(3, vz)
                    put(4, vx * vx)
                    put(5, vy * vy)
                    put(6, vz * vz)
                    put(7, vx * vy)
                    put(8, vx * vz)
                    put(9, vy * vz)
                pltpu.sync_copy(rowbuf, acc.at[sbuf.at[b]], add=True)
                return c2

            lax.fori_loop(0, BPC, blk_body, 0)

        return carry

    lax.fori_loop(0, TMAX, chunk_body, 0)
    _sc_epilogue(acc, mom_out, cid, sid)


@functools.partial(
    pl.kernel,
    out_type=jax.ShapeDtypeStruct((2, C, 16), jnp.float32),
    mesh=_MESH,
    scratch_types=[
        pltpu.VMEM_SHARED((C, 16), jnp.float32),
        pltpu.VMEM((PB, 5), jnp.float32),
        pltpu.VMEM((BPC, 128), jnp.int32),
        pltpu.VMEM((128, 16), jnp.float32),
        pltpu.VMEM((ZR, 16), jnp.float32),
        pltpu.VMEM((128, 16), jnp.float32),
    ],
    compiler_params=_SC_PARAMS,
)
def _sc_orient(data_hbm, seg2d_hbm, params_hbm, sc_out, acc, dbuf, sbuf,
               rowbuf, zbuf, prow):
    cid = lax.axis_index("c")
    sid = lax.axis_index("s")
    wid = cid * 16 + sid
    _sc_prologue(acc, rowbuf, zbuf, sid)
    iota = lax.iota(jnp.int32, 16)
    col0 = jnp.zeros((16,), jnp.int32)
    magic = jnp.full((16,), 0x5F3759DF, jnp.int32)
    one_i = jnp.full((16,), 1, jnp.int32)

    def chunk_body(t, carry):
        chunk = wid + NW * t

        @pl.when(chunk < NCHUNKS)
        def _():
            pltpu.sync_copy(data_hbm.at[pl.ds(chunk * PB, PB), :], dbuf)
            pltpu.sync_copy(seg2d_hbm.at[pl.ds(chunk * BPC, BPC), :], sbuf)

            def blk_body(b, c2):
                pltpu.sync_copy(params_hbm.at[sbuf.at[b]], prow)
                for g in range(8):
                    rows = iota + (b * 128 + g * 16)
                    rr = iota + g * 16
                    x = plsc.load_gather(dbuf, [rows, jnp.full((16,), 1, jnp.int32)])
                    y = plsc.load_gather(dbuf, [rows, jnp.full((16,), 2, jnp.int32)])
                    z = plsc.load_gather(dbuf, [rows, jnp.full((16,), 3, jnp.int32)])
                    cx = plsc.load_gather(prow, [rr, jnp.full((16,), 0, jnp.int32)])
                    cy = plsc.load_gather(prow, [rr, jnp.full((16,), 1, jnp.int32)])
                    cz = plsc.load_gather(prow, [rr, jnp.full((16,), 2, jnp.int32)])
                    vx = plsc.load_gather(prow, [rr, jnp.full((16,), 3, jnp.int32)])
                    vy = plsc.load_gather(prow, [rr, jnp.full((16,), 4, jnp.int32)])
                    vz = plsc.load_gather(prow, [rr, jnp.full((16,), 5, jnp.int32)])
                    xcx = x - cx
                    xcy = y - cy
                    xcz = z - cz
                    x0 = xcx * vx + xcy * vy + xcz * vz
                    d = xcx * xcx + xcy * xcy + xcz * xcz - x0 * x0
                    d = jnp.maximum(d, 0.0)
                    r = plsc.bitcast(magic - lax.shift_right_logical(
                        plsc.bitcast(d, jnp.int32), one_i), jnp.float32)
                    for _ in range(3):
                        h = 0.5 * d * r
                        r = r * (1.5 - h * r)
                    t_val = x0 * (d * r)
                    plsc.store_scatter(rowbuf, [rr, col0], t_val)
                pltpu.sync_copy(rowbuf, acc.at[sbuf.at[b]], add=True)
                return c2

            lax.fori_loop(0, BPC, blk_body, 0)

        return carry

    lax.fori_loop(0, TMAX, chunk_body, 0)
    _sc_epilogue(acc, sc_out, cid, sid)


def _jacobi_rot(app, aqq, apq):
    small = jnp.abs(apq) <= 1e-30
    apq_s = jnp.where(small, 1.0, apq)
    tau = (aqq - app) / (2.0 * apq_s)
    t = jnp.sign(tau) / (jnp.abs(tau) + jnp.sqrt(1.0 + tau * tau))
    t = jnp.where(tau == 0.0, 1.0, t)
    c = 1.0 / jnp.sqrt(1.0 + t * t)
    s = t * c
    c = jnp.where(small, 1.0, c)
    s = jnp.where(small, 0.0, s)
    return c, s


def _tc1_body(momref, featref, parref):
    m = [momref[0, j] + momref[1, j] for j in range(10)]
    n = m[0]
    n_safe = jnp.maximum(n, 1.0)
    sx, sy, sz = m[1], m[2], m[3]
    cx, cy, cz = sx / n_safe, sy / n_safe, sz / n_safe
    a00 = m[4] - sx * cx
    a11 = m[5] - sy * cy
    a22 = m[6] - sz * cz
    a01 = m[7] - sx * cy
    a02 = m[8] - sx * cz
    a12 = m[9] - sy * cz
    safe = n >= 2.0
    a00 = jnp.where(safe, a00, 1.0)
    a11 = jnp.where(safe, a11, 2.0)
    a22 = jnp.where(safe, a22, 3.0)
    a01 = jnp.where(safe, a01, 0.0)
    a02 = jnp.where(safe, a02, 0.0)
    a12 = jnp.where(safe, a12, 0.0)
    g00, g01, g02, g11, g12, g22 = a00, a01, a02, a11, a12, a22

    one = jnp.ones_like(a00)
    zero = jnp.zeros_like(a00)
    v00, v01, v02 = one, zero, zero
    v10, v11, v12 = zero, one, zero
    v20, v21, v22 = zero, zero, one

    for _ in range(4):
        c, s = _jacobi_rot(a00, a11, a01)
        a00, a11 = (c * c * a00 - 2 * s * c * a01 + s * s * a11,
                    s * s * a00 + 2 * s * c * a01 + c * c * a11)
        a02, a12 = c * a02 - s * a12, s * a02 + c * a12
        a01 = zero
        v00, v01 = c * v00 - s * v01, s * v00 + c * v01
        v10, v11 = c * v10 - s * v11, s * v10 + c * v11
        v20, v21 = c * v20 - s * v21, s * v20 + c * v21

        c, s = _jacobi_rot(a00, a22, a02)
        a00, a22 = (c * c * a00 - 2 * s * c * a02 + s * s * a22,
                    s * s * a00 + 2 * s * c * a02 + c * c * a22)
        a01, a12 = c * a01 - s * a12, s * a01 + c * a12
        a02 = zero
        v00, v02 = c * v00 - s * v02, s * v00 + c * v02
        v10, v12 = c * v10 - s * v12, s * v10 + c * v12
        v20, v22 = c * v20 - s * v22, s * v20 + c * v22

        c, s = _jacobi_rot(a11, a22, a12)
        a11, a22 = (c * c * a11 - 2 * s * c * a12 + s * s * a22,
                    s * s * a11 + 2 * s * c * a12 + c * c * a22)
        a01, a02 = c * a01 - s * a02, s * a01 + c * a02
        a12 = zero
        v01, v02 = c * v01 - s * v02, s * v01 + c * v02
        v11, v12 = c * v11 - s * v12, s * v11 + c * v12
        v21, v22 = c * v21 - s * v22, s * v21 + c * v22

    d0, d1, d2 = a00, a11, a22
    w2 = jnp.maximum(jnp.maximum(d0, d1), d2)
    w0 = jnp.minimum(jnp.minimum(d0, d1), d2)
    w1 = d0 + d1 + d2 - w2 - w0
    is0 = (d0 >= d1) & (d0 >= d2)
    is1 = jnp.logical_not(is0) & (d1 >= d2)
    v0x = jnp.where(is0, v00, jnp.where(is1, v01, v02))
    v0y = jnp.where(is0, v10, jnp.where(is1, v11, v12))
    v0z = jnp.where(is0, v20, jnp.where(is1, v21, v22))

    w2s = jnp.where(w2 != 0.0, w2, 1.0)
    dirwt = 1.0 - w1 / w2s

    feats = [
        jnp.where(safe, cx, sx),
        jnp.where(safe, cy, sy),
        jnp.where(safe, cz, sz),
        jnp.where(safe, g00 / w2s, 0.0),
        jnp.where(safe, g01 / w2s, 0.0),
        jnp.where(safe, g02 / w2s, 0.0),
        jnp.where(safe, g01 / w2s, 0.0),
        jnp.where(safe, g11 / w2s, 0.0),
        jnp.where(safe, g12 / w2s, 0.0),
        jnp.where(safe, g02 / w2s, 0.0),
        jnp.where(safe, g12 / w2s, 0.0),
        jnp.where(safe, g22 / w2s, 0.0),
        jnp.where(safe, dirwt * v0x, 0.0),
        jnp.where(safe, dirwt * v0y, 0.0),
        jnp.where(safe, dirwt * v0z, 0.0),
        n,
    ]
    for j in range(16):
        featref[j] = feats[j]
    pars = [cx, cy, cz, v0x, v0y, v0z]
    for j in range(6):
        parref[j] = pars[j]
    for j in range(6, 16):
        parref[j] = zero


def _tc1(momT):
    return pl.pallas_call(
        _tc1_body,
        grid=(GRID,),
        in_specs=[pl.BlockSpec((2, 16, 8, 128), lambda i: (0, 0, i, 0))],
        out_specs=[pl.BlockSpec((16, 8, 128), lambda i: (0, i, 0)),
                   pl.BlockSpec((16, 8, 128), lambda i: (0, i, 0))],
        out_shape=[jax.ShapeDtypeStruct((16, G, 128), jnp.float32),
                   jax.ShapeDtypeStruct((16, G, 128), jnp.float32)],
    )(momT)


def _tc2_body(featref, scref, outref):
    sc = scref[0] + scref[1]
    n = featref[15]
    flip = (n >= 2.0) & (sc < 0.0)
    fac = jnp.where(flip, -1.0, 1.0)
    for j in range(12):
        outref[j] = featref[j]
    for j in (12, 13, 14):
        outref[j] = featref[j] * fac
    outref[15] = n


def _tc2(feats0T, scs):
    return pl.pallas_call(
        _tc2_body,
        grid=(GRID,),
        in_specs=[pl.BlockSpec((16, 8, 128), lambda i: (0, i, 0)),
                  pl.BlockSpec((2, 8, 128), lambda i: (0, i, 0))],
        out_specs=pl.BlockSpec((16, 8, 128), lambda i: (0, i, 0)),
        out_shape=jax.ShapeDtypeStruct((16, G, 128), jnp.float32),
    )(feats0T, scs)


def kernel(data, segment_ids):
    seg = segment_ids.astype(jnp.int32)
    seg2d = seg.reshape(NB, 128)

    mom = _sc_moments(data, seg2d)
    momT = jnp.pad(jnp.transpose(mom, (0, 2, 1)),
                   ((0, 0), (0, 0), (0, CPAD - C))).reshape(2, 16, G, 128)
    feats0T, paramsT = _tc1(momT)
    params = paramsT.reshape(16, CPAD)[:, :C].T

    sc_acc = _sc_orient(data, seg2d, params)
    scs = jnp.pad(sc_acc[:, :, 0], ((0, 0), (0, CPAD - C))).reshape(2, G, 128)
    outT = _tc2(feats0T, scs)
    return outT.reshape(16, CPAD)[:, :C].T


# speedup vs baseline: 219.6174x; 3.1899x over previous
import functools

import jax
import jax.numpy as jnp
from jax import lax
from jax.experimental import pallas as pl
from jax.experimental.pallas import tpu as pltpu
from jax.experimental.pallas import tpu_sc as plsc

N = 1_600_000
C = 50_000

NB = N // 128
BPC = 20
PB = BPC * 128
NCHUNKS = N // PB
NW = 32
TMAX = (NCHUNKS + NW - 1) // NW

CPAD = 50_176
G = CPAD // 128
GRID = G // 8
RPT = CPAD // 16
ZR = RPT // 4

_MESH = plsc.VectorSubcoreMesh(core_axis_name="c", subcore_axis_name="s")
_SC_PARAMS = pltpu.CompilerParams(use_tc_tiling_on_sc=False,
                                  needs_layout_passes=False)


def _zero_rows(ref, nrows):
    zero16 = jnp.zeros((16,), jnp.float32)

    def body(i, carry):
        ref[i, :] = zero16
        return carry

    lax.fori_loop(0, nrows, body, 0)


def _sc_prologue(acc, rowbuf, zbuf, sid):
    _zero_rows(rowbuf, 128)
    _zero_rows(zbuf, ZR)
    base = sid * RPT
    for r in range(4):
        pltpu.sync_copy(zbuf, acc.at[pl.ds(base + r * ZR, ZR), :])


@functools.partial(
    pl.kernel,
    out_type=jax.ShapeDtypeStruct((2, 10, CPAD), jnp.float32),
    mesh=_MESH,
    scratch_types=[
        pltpu.VMEM_SHARED((CPAD, 16), jnp.float32),
        pltpu.VMEM((BPC, 128), jnp.float32),
        pltpu.VMEM((BPC, 128), jnp.float32),
        pltpu.VMEM((BPC, 128), jnp.float32),
        pltpu.VMEM((BPC, 128), jnp.int32),
        pltpu.VMEM((128, 16), jnp.float32),
        pltpu.VMEM((ZR, 16), jnp.float32),
        pltpu.VMEM((10, ZR), jnp.float32),
    ],
    compiler_params=_SC_PARAMS,
)
def _sc_moments(x_hbm, y_hbm, z_hbm, seg_hbm, mom_out,
                acc, xbuf, ybuf, zbuf, sbuf, rowbuf, zrow, trows):
    cid = lax.axis_index("c")
    sid = lax.axis_index("s")
    wid = cid * 16 + sid
    _sc_prologue(acc, rowbuf, zrow, sid)
    plsc.subcore_barrier()
    iota = lax.iota(jnp.int32, 16)
    ones = jnp.full((16,), 1.0, jnp.float32)
    for g in range(8):
        plsc.store_scatter(rowbuf, [iota + g * 16, jnp.zeros((16,), jnp.int32)],
                           ones)

    def chunk_body(t, carry):
        chunk = wid + NW * t

        @pl.when(chunk < NCHUNKS)
        def _():
            blk0 = chunk * BPC
            pltpu.sync_copy(x_hbm.at[pl.ds(blk0, BPC), :], xbuf)
            pltpu.sync_copy(y_hbm.at[pl.ds(blk0, BPC), :], ybuf)
            pltpu.sync_copy(z_hbm.at[pl.ds(blk0, BPC), :], zbuf)
            pltpu.sync_copy(seg_hbm.at[pl.ds(blk0, BPC), :], sbuf)

            def blk_body(b, c2):
                for g in range(8):
                    sl = pl.ds(g * 16, 16)
                    rr = iota + g * 16
                    vx = xbuf[b, sl]
                    vy = ybuf[b, sl]
                    vz = zbuf[b, sl]

                    def put(col, val):
                        plsc.store_scatter(
                            rowbuf, [rr, jnp.full((16,), col, jnp.int32)], val)

                    put(1, vx)
                    put(2, vy)
                    put(3, vz)
                    put(4, vx * vx)
                    put(5, vy * vy)
                    put(6, vz * vz)
                    put(7, vx * vy)
                    put(8, vx * vz)
                    put(9, vy * vz)
                pltpu.sync_copy(rowbuf, acc.at[sbuf.at[b]], add=True)
                return c2

            lax.fori_loop(0, BPC, blk_body, 0)

        return carry

    lax.fori_loop(0, TMAX, chunk_body, 0)
    plsc.subcore_barrier()
    for ch in range(4):
        base = sid * RPT + ch * ZR
        pltpu.sync_copy(acc.at[pl.ds(base, ZR), :], zrow)

        def grp_body(g2, c3):
            rows = iota + g2 * 16
            for j in range(10):
                v = plsc.load_gather(zrow, [rows, jnp.full((16,), j, jnp.int32)])
                trows[j, pl.ds(g2 * 16, 16)] = v
            return c3

        lax.fori_loop(0, ZR // 16, grp_body, 0)
        pltpu.sync_copy(trows, mom_out.at[cid, :, pl.ds(base, ZR)])


@functools.partial(
    pl.kernel,
    out_type=jax.ShapeDtypeStruct((2, CPAD), jnp.float32),
    mesh=_MESH,
    scratch_types=[
        pltpu.VMEM_SHARED((CPAD, 16), jnp.float32),
        pltpu.VMEM_SHARED((C, 16), jnp.float32),
        pltpu.VMEM((BPC, 128), jnp.float32),
        pltpu.VMEM((BPC, 128), jnp.float32),
        pltpu.VMEM((BPC, 128), jnp.float32),
        pltpu.VMEM((BPC, 128), jnp.int32),
        pltpu.VMEM((128, 16), jnp.float32),
        pltpu.VMEM((ZR, 16), jnp.float32),
        pltpu.VMEM((128, 16), jnp.float32),
        pltpu.VMEM((ZR,), jnp.float32),
    ],
    compiler_params=_SC_PARAMS,
)
def _sc_orient(x_hbm, y_hbm, z_hbm, seg_hbm, params_hbm, sc_out,
               acc, ptab, xbuf, ybuf, zbuf, sbuf, rowbuf, zrow, prow, srow):
    cid = lax.axis_index("c")
    sid = lax.axis_index("s")
    wid = cid * 16 + sid
    _sc_prologue(acc, rowbuf, zrow, sid)
    pltpu.sync_copy(params_hbm.at[pl.ds(sid * 3125, 3125), :],
                    ptab.at[pl.ds(sid * 3125, 3125), :])
    plsc.subcore_barrier()
    iota = lax.iota(jnp.int32, 16)
    col0 = jnp.zeros((16,), jnp.int32)
    magic = jnp.full((16,), 0x5F3759DF, jnp.int32)
    one_i = jnp.full((16,), 1, jnp.int32)

    def chunk_body(t, carry):
        chunk = wid + NW * t

        @pl.when(chunk < NCHUNKS)
        def _():
            blk0 = chunk * BPC
            pltpu.sync_copy(x_hbm.at[pl.ds(blk0, BPC), :], xbuf)
            pltpu.sync_copy(y_hbm.at[pl.ds(blk0, BPC), :], ybuf)
            pltpu.sync_copy(z_hbm.at[pl.ds(blk0, BPC), :], zbuf)
            pltpu.sync_copy(seg_hbm.at[pl.ds(blk0, BPC), :], sbuf)

            def blk_body(b, c2):
                pltpu.sync_copy(ptab.at[sbuf.at[b]], prow)
                for g in range(8):
                    sl = pl.ds(g * 16, 16)
                    rr = iota + g * 16
                    x = xbuf[b, sl]
                    y = ybuf[b, sl]
                    z = zbuf[b, sl]
                    cx = plsc.load_gather(prow, [rr, jnp.full((16,), 0, jnp.int32)])
                    cy = plsc.load_gather(prow, [rr, jnp.full((16,), 1, jnp.int32)])
                    cz = plsc.load_gather(prow, [rr, jnp.full((16,), 2, jnp.int32)])
                    vx = plsc.load_gather(prow, [rr, jnp.full((16,), 3, jnp.int32)])
                    vy = plsc.load_gather(prow, [rr, jnp.full((16,), 4, jnp.int32)])
                    vz = plsc.load_gather(prow, [rr, jnp.full((16,), 5, jnp.int32)])
                    xcx = x - cx
                    xcy = y - cy
                    xcz = z - cz
                    x0 = xcx * vx + xcy * vy + xcz * vz
                    d = xcx * xcx + xcy * xcy + xcz * xcz - x0 * x0
                    d = jnp.maximum(d, 0.0)
                    r = plsc.bitcast(magic - lax.shift_right_logical(
                        plsc.bitcast(d, jnp.int32), one_i), jnp.float32)
                    for _ in range(3):
                        h = 0.5 * d * r
                        r = r * (1.5 - h * r)
                    t_val = x0 * (d * r)
                    plsc.store_scatter(rowbuf, [rr, col0], t_val)
                pltpu.sync_copy(rowbuf, acc.at[sbuf.at[b]], add=True)
                return c2

            lax.fori_loop(0, BPC, blk_body, 0)

        return carry

    lax.fori_loop(0, TMAX, chunk_body, 0)
    plsc.subcore_barrier()
    for ch in range(4):
        base = sid * RPT + ch * ZR
        pltpu.sync_copy(acc.at[pl.ds(base, ZR), :], zrow)

        def grp_body(g2, c3):
            v = plsc.load_gather(zrow, [iota + g2 * 16, col0])
            srow[pl.ds(g2 * 16, 16)] = v
            return c3

        lax.fori_loop(0, ZR // 16, grp_body, 0)
        pltpu.sync_copy(srow, sc_out.at[cid, pl.ds(base, ZR)])


def _jacobi_rot(app, aqq, apq):
    small = jnp.abs(apq) <= 1e-30
    apq_s = jnp.where(small, 1.0, apq)
    tau = (aqq - app) / (2.0 * apq_s)
    t = jnp.sign(tau) / (jnp.abs(tau) + jnp.sqrt(1.0 + tau * tau))
    t = jnp.where(tau == 0.0, 1.0, t)
    c = 1.0 / jnp.sqrt(1.0 + t * t)
    s = t * c
    c = jnp.where(small, 1.0, c)
    s = jnp.where(small, 0.0, s)
    return c, s


def _tc1_body(momref, featref, parref):
    m = [momref[0, j] + momref[1, j] for j in range(10)]
    n = m[0]
    n_safe = jnp.maximum(n, 1.0)
    sx, sy, sz = m[1], m[2], m[3]
    cx, cy, cz = sx / n_safe, sy / n_safe, sz / n_safe
    a00 = m[4] - sx * cx
    a11 = m[5] - sy * cy
    a22 = m[6] - sz * cz
    a01 = m[7] - sx * cy
    a02 = m[8] - sx * cz
    a12 = m[9] - sy * cz
    safe = n >= 2.0
    a00 = jnp.where(safe, a00, 1.0)
    a11 = jnp.where(safe, a11, 2.0)
    a22 = jnp.where(safe, a22, 3.0)
    a01 = jnp.where(safe, a01, 0.0)
    a02 = jnp.where(safe, a02, 0.0)
    a12 = jnp.where(safe, a12, 0.0)
    g00, g01, g02, g11, g12, g22 = a00, a01, a02, a11, a12, a22

    one = jnp.ones_like(a00)
    zero = jnp.zeros_like(a00)
    v00, v01, v02 = one, zero, zero
    v10, v11, v12 = zero, one, zero
    v20, v21, v22 = zero, zero, one

    for _ in range(4):
        c, s = _jacobi_rot(a00, a11, a01)
        a00, a11 = (c * c * a00 - 2 * s * c * a01 + s * s * a11,
                    s * s * a00 + 2 * s * c * a01 + c * c * a11)
        a02, a12 = c * a02 - s * a12, s * a02 + c * a12
        a01 = zero
        v00, v01 = c * v00 - s * v01, s * v00 + c * v01
        v10, v11 = c * v10 - s * v11, s * v10 + c * v11
        v20, v21 = c * v20 - s * v21, s * v20 + c * v21

        c, s = _jacobi_rot(a00, a22, a02)
        a00, a22 = (c * c * a00 - 2 * s * c * a02 + s * s * a22,
                    s * s * a00 + 2 * s * c * a02 + c * c * a22)
        a01, a12 = c * a01 - s * a12, s * a01 + c * a12
        a02 = zero
        v00, v02 = c * v00 - s * v02, s * v00 + c * v02
        v10, v12 = c * v10 - s * v12, s * v10 + c * v12
        v20, v22 = c * v20 - s * v22, s * v20 + c * v22

        c, s = _jacobi_rot(a11, a22, a12)
        a11, a22 = (c * c * a11 - 2 * s * c * a12 + s * s * a22,
                    s * s * a11 + 2 * s * c * a12 + c * c * a22)
        a01, a02 = c * a01 - s * a02, s * a01 + c * a02
        a12 = zero
        v01, v02 = c * v01 - s * v02, s * v01 + c * v02
        v11, v12 = c * v11 - s * v12, s * v11 + c * v12
        v21, v22 = c * v21 - s * v22, s * v21 + c * v22

    d0, d1, d2 = a00, a11, a22
    w2 = jnp.maximum(jnp.maximum(d0, d1), d2)
    w0 = jnp.minimum(jnp.minimum(d0, d1), d2)
    w1 = d0 + d1 + d2 - w2 - w0
    is0 = (d0 >= d1) & (d0 >= d2)
    is1 = jnp.logical_not(is0) & (d1 >= d2)
    v0x = jnp.where(is0, v00, jnp.where(is1, v01, v02))
    v0y = jnp.where(is0, v10, jnp.where(is1, v11, v12))
    v0z = jnp.where(is0, v20, jnp.where(is1, v21, v22))

    w2s = jnp.where(w2 != 0.0, w2, 1.0)
    dirwt = 1.0 - w1 / w2s

    feats = [
        jnp.where(safe, cx, sx),
        jnp.where(safe, cy, sy),
        jnp.where(safe, cz, sz),
        jnp.where(safe, g00 / w2s, 0.0),
        jnp.where(safe, g01 / w2s, 0.0),
        jnp.where(safe, g02 / w2s, 0.0),
        jnp.where(safe, g01 / w2s, 0.0),
        jnp.where(safe, g11 / w2s, 0.0),
        jnp.where(safe, g12 / w2s, 0.0),
        jnp.where(safe, g02 / w2s, 0.0),
        jnp.where(safe, g12 / w2s, 0.0),
        jnp.where(safe, g22 / w2s, 0.0),
        jnp.where(safe, dirwt * v0x, 0.0),
        jnp.where(safe, dirwt * v0y, 0.0),
        jnp.where(safe, dirwt * v0z, 0.0),
        n,
    ]
    for j in range(16):
        featref[j] = feats[j]
    pars = [cx, cy, cz, v0x, v0y, v0z]
    for j in range(6):
        parref[j] = pars[j]
    for j in range(6, 16):
        parref[j] = zero


def _tc1(momT):
    return pl.pallas_call(
        _tc1_body,
        grid=(GRID,),
        in_specs=[pl.BlockSpec((2, 10, 8, 128), lambda i: (0, 0, i, 0))],
        out_specs=[pl.BlockSpec((16, 8, 128), lambda i: (0, i, 0)),
                   pl.BlockSpec((16, 8, 128), lambda i: (0, i, 0))],
        out_shape=[jax.ShapeDtypeStruct((16, G, 128), jnp.float32),
                   jax.ShapeDtypeStruct((16, G, 128), jnp.float32)],
    )(momT)


def _tc2_body(featref, scref, outref):
    sc = scref[0] + scref[1]
    n = featref[15]
    flip = (n >= 2.0) & (sc < 0.0)
    fac = jnp.where(flip, -1.0, 1.0)
    for j in range(12):
        outref[j] = featref[j]
    for j in (12, 13, 14):
        outref[j] = featref[j] * fac
    outref[15] = n


def _tc2(feats0T, scs):
    return pl.pallas_call(
        _tc2_body,
        grid=(GRID,),
        in_specs=[pl.BlockSpec((16, 8, 128), lambda i: (0, i, 0)),
                  pl.BlockSpec((2, 8, 128), lambda i: (0, i, 0))],
        out_specs=pl.BlockSpec((16, 8, 128), lambda i: (0, i, 0)),
        out_shape=jax.ShapeDtypeStruct((16, G, 128), jnp.float32),
    )(feats0T, scs)


def kernel(data, segment_ids):
    seg2d = segment_ids.astype(jnp.int32).reshape(NB, 128)
    xs = data[:, 1].reshape(NB, 128)
    ys = data[:, 2].reshape(NB, 128)
    zs = data[:, 3].reshape(NB, 128)

    mom = _sc_moments(xs, ys, zs, seg2d)
    momT = mom.reshape(2, 10, G, 128)
    feats0T, paramsT = _tc1(momT)
    params = paramsT.reshape(16, CPAD)[:, :C].T

    sc_acc = _sc_orient(xs, ys, zs, seg2d, params)
    scs = sc_acc.reshape(2, G, 128)
    outT = _tc2(feats0T, scs)
    return outT.reshape(16, CPAD)[:, :C].T


# speedup vs baseline: 356.9462x; 1.6253x over previous
import functools

import jax
import jax.numpy as jnp
from jax import lax
from jax.experimental import pallas as pl
from jax.experimental.pallas import tpu as pltpu
from jax.experimental.pallas import tpu_sc as plsc

N = 1_600_000
C = 50_000

NB = N // 128
BPC = 20
PB = BPC * 128
NCHUNKS = N // PB
NW = 32
TMAX = (NCHUNKS + NW - 1) // NW

CPAD = 50_176
G = CPAD // 128
GRID = G // 8
RPT = CPAD // 16
ZR = RPT // 4

_MESH = plsc.VectorSubcoreMesh(core_axis_name="c", subcore_axis_name="s")
_SC_PARAMS = pltpu.CompilerParams(use_tc_tiling_on_sc=False,
                                  needs_layout_passes=False)


def _zero_rows(ref, nrows):
    zero16 = jnp.zeros((16,), jnp.float32)

    def body(i, carry):
        ref[i, :] = zero16
        return carry

    lax.fori_loop(0, nrows, body, 0)


def _sc_prologue(acc, rowbuf, zbuf, sid):
    _zero_rows(rowbuf, 128)
    _zero_rows(zbuf, ZR)
    base = sid * RPT
    for r in range(4):
        pltpu.sync_copy(zbuf, acc.at[pl.ds(base + r * ZR, ZR), :])


def _fire_inputs(x_hbm, y_hbm, z_hbm, seg_hbm, xbuf, ybuf, zbuf, sbuf,
                 slot, chunk, insem):
    blk0 = chunk * BPC
    pltpu.async_copy(x_hbm.at[pl.ds(blk0, BPC), :], xbuf.at[slot], insem)
    pltpu.async_copy(y_hbm.at[pl.ds(blk0, BPC), :], ybuf.at[slot], insem)
    pltpu.async_copy(z_hbm.at[pl.ds(blk0, BPC), :], zbuf.at[slot], insem)
    pltpu.async_copy(seg_hbm.at[pl.ds(blk0, BPC), :], sbuf.at[slot], insem)


def _drain_inputs(x_hbm, y_hbm, z_hbm, seg_hbm, xbuf, ybuf, zbuf, sbuf,
                  slot, insem):
    pltpu.make_async_copy(x_hbm.at[pl.ds(0, BPC), :], xbuf.at[slot], insem).wait()
    pltpu.make_async_copy(y_hbm.at[pl.ds(0, BPC), :], ybuf.at[slot], insem).wait()
    pltpu.make_async_copy(z_hbm.at[pl.ds(0, BPC), :], zbuf.at[slot], insem).wait()
    pltpu.make_async_copy(seg_hbm.at[pl.ds(0, BPC), :], sbuf.at[slot], insem).wait()


@functools.partial(
    pl.kernel,
    out_type=jax.ShapeDtypeStruct((2, 10, CPAD), jnp.float32),
    mesh=_MESH,
    scratch_types=[
        pltpu.VMEM_SHARED((CPAD, 16), jnp.float32),
        pltpu.VMEM((2, BPC, 128), jnp.float32),
        pltpu.VMEM((2, BPC, 128), jnp.float32),
        pltpu.VMEM((2, BPC, 128), jnp.float32),
        pltpu.VMEM((2, BPC, 128), jnp.int32),
        pltpu.VMEM((2, 128, 16), jnp.float32),
        pltpu.VMEM((ZR, 16), jnp.float32),
        pltpu.VMEM((10, ZR), jnp.float32),
        pltpu.SemaphoreType.DMA,
        pltpu.SemaphoreType.DMA,
        pltpu.SemaphoreType.DMA,
    ],
    compiler_params=_SC_PARAMS,
)
def _sc_moments(x_hbm, y_hbm, z_hbm, seg_hbm, mom_out,
                acc, xbuf, ybuf, zbuf, sbuf, rowbuf, zrow, trows,
                insem, scsem0, scsem1):
    cid = lax.axis_index("c")
    sid = lax.axis_index("s")
    wid = cid * 16 + sid
    _zero_rows(rowbuf.at[0], 128)
    _zero_rows(rowbuf.at[1], 128)
    _zero_rows(zrow, ZR)
    base0 = sid * RPT
    for r in range(4):
        pltpu.sync_copy(zrow, acc.at[pl.ds(base0 + r * ZR, ZR), :])
    plsc.subcore_barrier()
    iota = lax.iota(jnp.int32, 16)
    ones = jnp.full((16,), 1.0, jnp.float32)
    for r in range(2):
        for g in range(8):
            plsc.store_scatter(rowbuf.at[r],
                               [iota + g * 16, jnp.zeros((16,), jnp.int32)],
                               ones)
    scsems = (scsem0, scsem1)
    drain_dst = (rowbuf.at[0], rowbuf.at[1])

    _fire_inputs(x_hbm, y_hbm, z_hbm, seg_hbm, xbuf, ybuf, zbuf, sbuf,
                 0, wid, insem)

    def chunk_body(t, carry):
        chunk = wid + NW * t
        slot = lax.rem(t, 2)

        @pl.when(chunk < NCHUNKS)
        def _():
            _drain_inputs(x_hbm, y_hbm, z_hbm, seg_hbm, xbuf, ybuf, zbuf,
                          sbuf, slot, insem)
            nxt = chunk + NW

            @pl.when(nxt < NCHUNKS)
            def _():
                _fire_inputs(x_hbm, y_hbm, z_hbm, seg_hbm, xbuf, ybuf, zbuf,
                             sbuf, 1 - slot, nxt, insem)

            xb = xbuf.at[slot]
            yb = ybuf.at[slot]
            zb = zbuf.at[slot]
            sb = sbuf.at[slot]

            def pair_body(p, c2):
                for r in range(2):
                    b = 2 * p + r
                    rb = rowbuf.at[r]

                    @pl.when((p > 0) | (t > 0))
                    def _():
                        pltpu.make_async_copy(
                            x_hbm.at[pl.ds(0, 128), pl.ds(0, 16)], rb,
                            scsems[r]).wait()
                    for g in range(8):
                        sl = pl.ds(g * 16, 16)
                        rr = iota + g * 16
                        vx = xb[b, sl]
                        vy = yb[b, sl]
                        vz = zb[b, sl]

                        def put(col, val):
                            plsc.store_scatter(
                                rb, [rr, jnp.full((16,), col, jnp.int32)], val)

                        put(1, vx)
                        put(2, vy)
                        put(3, vz)
                        put(4, vx * vx)
                        put(5, vy * vy)
                        put(6, vz * vz)
                        put(7, vx * vy)
                        put(8, vx * vz)
                        put(9, vy * vz)
                    pltpu.async_copy(rb, acc.at[sb.at[b]], scsems[r], add=True)
                return c2

            lax.fori_loop(0, BPC // 2, pair_body, 0)

        return carry

    lax.fori_loop(0, TMAX, chunk_body, 0)
    for r in range(2):
        pltpu.make_async_copy(x_hbm.at[pl.ds(0, 128), pl.ds(0, 16)],
                              drain_dst[r], scsems[r]).wait()
    plsc.subcore_barrier()
    for ch in range(4):
        base = sid * RPT + ch * ZR
        pltpu.sync_copy(acc.at[pl.ds(base, ZR), :], zrow)

        def grp_body(g2, c3):
            rows = iota + g2 * 16
            for j in range(10):
                v = plsc.load_gather(zrow, [rows, jnp.full((16,), j, jnp.int32)])
                trows[j, pl.ds(g2 * 16, 16)] = v
            return c3

        lax.fori_loop(0, ZR // 16, grp_body, 0)
        pltpu.sync_copy(trows, mom_out.at[cid, :, pl.ds(base, ZR)])


@functools.partial(
    pl.kernel,
    out_type=jax.ShapeDtypeStruct((2, CPAD), jnp.float32),
    mesh=_MESH,
    scratch_types=[
        pltpu.VMEM_SHARED((CPAD, 16), jnp.float32),
        pltpu.VMEM_SHARED((C, 8), jnp.float32),
        pltpu.VMEM((2, BPC, 128), jnp.float32),
        pltpu.VMEM((2, BPC, 128), jnp.float32),
        pltpu.VMEM((2, BPC, 128), jnp.float32),
        pltpu.VMEM((2, BPC, 128), jnp.int32),
        pltpu.VMEM((2, 128, 16), jnp.float32),
        pltpu.VMEM((ZR, 16), jnp.float32),
        pltpu.VMEM((2, 128, 8), jnp.float32),
        pltpu.VMEM((ZR,), jnp.float32),
        pltpu.SemaphoreType.DMA,
        pltpu.SemaphoreType.DMA,
        pltpu.SemaphoreType.DMA,
        pltpu.SemaphoreType.DMA,
        pltpu.SemaphoreType.DMA,
    ],
    compiler_params=_SC_PARAMS,
)
def _sc_orient(x_hbm, y_hbm, z_hbm, seg_hbm, params_hbm, sc_out,
               acc, ptab, xbuf, ybuf, zbuf, sbuf, rowbuf, zrow, prow, srow,
               insem, scsem0, scsem1, gsem0, gsem1):
    cid = lax.axis_index("c")
    sid = lax.axis_index("s")
    wid = cid * 16 + sid
    _zero_rows(rowbuf.at[0], 128)
    _zero_rows(rowbuf.at[1], 128)
    _zero_rows(zrow, ZR)
    base0 = sid * RPT
    for r in range(4):
        pltpu.sync_copy(zrow, acc.at[pl.ds(base0 + r * ZR, ZR), :])
    pltpu.sync_copy(params_hbm.at[pl.ds(sid * 3125, 3125), :],
                    ptab.at[pl.ds(sid * 3125, 3125), :])
    plsc.subcore_barrier()
    iota = lax.iota(jnp.int32, 16)
    col0 = jnp.zeros((16,), jnp.int32)
    magic = jnp.full((16,), 0x5F3759DF, jnp.int32)
    one_i = jnp.full((16,), 1, jnp.int32)
    scsems = (scsem0, scsem1)
    gsems = (gsem0, gsem1)

    _fire_inputs(x_hbm, y_hbm, z_hbm, seg_hbm, xbuf, ybuf, zbuf, sbuf,
                 0, wid, insem)

    def chunk_body(t, carry):
        chunk = wid + NW * t
        slot = lax.rem(t, 2)

        @pl.when(chunk < NCHUNKS)
        def _():
            _drain_inputs(x_hbm, y_hbm, z_hbm, seg_hbm, xbuf, ybuf, zbuf,
                          sbuf, slot, insem)
            nxt = chunk + NW

            @pl.when(nxt < NCHUNKS)
            def _():
                _fire_inputs(x_hbm, y_hbm, z_hbm, seg_hbm, xbuf, ybuf, zbuf,
                             sbuf, 1 - slot, nxt, insem)

            xb = xbuf.at[slot]
            yb = ybuf.at[slot]
            zb = zbuf.at[slot]
            sb = sbuf.at[slot]
            pltpu.async_copy(ptab.at[sb.at[0]], prow.at[0], gsem0)

            def pair_body(p, c2):
                for r in range(2):
                    b = 2 * p + r
                    rb = rowbuf.at[r]
                    pb = prow.at[r]
                    pltpu.make_async_copy(
                        params_hbm.at[pl.ds(0, 128), :], pb,
                        gsems[r]).wait()

                    @pl.when(b + 1 < BPC)
                    def _():
                        pltpu.async_copy(ptab.at[sb.at[b + 1]],
                                         prow.at[1 - r], gsems[1 - r])

                    @pl.when((p > 0) | (t > 0))
                    def _():
                        pltpu.make_async_copy(
                            x_hbm.at[pl.ds(0, 128), pl.ds(0, 16)], rb,
                            scsems[r]).wait()
                    for g in range(8):
                        sl = pl.ds(g * 16, 16)
                        rr = iota + g * 16
                        x = xb[b, sl]
                        y = yb[b, sl]
                        z = zb[b, sl]
                        cx = plsc.load_gather(pb, [rr, jnp.full((16,), 0, jnp.int32)])
                        cy = plsc.load_gather(pb, [rr, jnp.full((16,), 1, jnp.int32)])
                        cz = plsc.load_gather(pb, [rr, jnp.full((16,), 2, jnp.int32)])
                        vx = plsc.load_gather(pb, [rr, jnp.full((16,), 3, jnp.int32)])
                        vy = plsc.load_gather(pb, [rr, jnp.full((16,), 4, jnp.int32)])
                        vz = plsc.load_gather(pb, [rr, jnp.full((16,), 5, jnp.int32)])
                        xcx = x - cx
                        xcy = y - cy
                        xcz = z - cz
                        x0 = xcx * vx + xcy * vy + xcz * vz
                        d = xcx * xcx + xcy * xcy + xcz * xcz - x0 * x0
                        d = jnp.maximum(d, 0.0)
                        rv = plsc.bitcast(magic - lax.shift_right_logical(
                            plsc.bitcast(d, jnp.int32), one_i), jnp.float32)
                        for _ in range(3):
                            h = 0.5 * d * rv
                            rv = rv * (1.5 - h * rv)
                        t_val = x0 * (d * rv)
                        plsc.store_scatter(rb, [rr, col0], t_val)
                    pltpu.async_copy(rb, acc.at[sb.at[b]], scsems[r], add=True)
                return c2

            lax.fori_loop(0, BPC // 2, pair_body, 0)

        return carry

    lax.fori_loop(0, TMAX, chunk_body, 0)
    for r in range(2):
        pltpu.make_async_copy(x_hbm.at[pl.ds(0, 128), pl.ds(0, 16)],
                              rowbuf.at[r], scsems[r]).wait()
    plsc.subcore_barrier()
    for ch in range(4):
        base = sid * RPT + ch * ZR
        pltpu.sync_copy(acc.at[pl.ds(base, ZR), :], zrow)

        def grp_body(g2, c3):
            v = plsc.load_gather(zrow, [iota + g2 * 16, col0])
            srow[pl.ds(g2 * 16, 16)] = v
            return c3

        lax.fori_loop(0, ZR // 16, grp_body, 0)
        pltpu.sync_copy(srow, sc_out.at[cid, pl.ds(base, ZR)])


def _jacobi_rot(app, aqq, apq):
    small = jnp.abs(apq) <= 1e-30
    apq_s = jnp.where(small, 1.0, apq)
    tau = (aqq - app) / (2.0 * apq_s)
    t = jnp.sign(tau) / (jnp.abs(tau) + jnp.sqrt(1.0 + tau * tau))
    t = jnp.where(tau == 0.0, 1.0, t)
    c = 1.0 / jnp.sqrt(1.0 + t * t)
    s = t * c
    c = jnp.where(small, 1.0, c)
    s = jnp.where(small, 0.0, s)
    return c, s


def _tc1_body(momref, featref, parref):
    m = [momref[0, j] + momref[1, j] for j in range(10)]
    n = m[0]
    n_safe = jnp.maximum(n, 1.0)
    sx, sy, sz = m[1], m[2], m[3]
    cx, cy, cz = sx / n_safe, sy / n_safe, sz / n_safe
    a00 = m[4] - sx * cx
    a11 = m[5] - sy * cy
    a22 = m[6] - sz * cz
    a01 = m[7] - sx * cy
    a02 = m[8] - sx * cz
    a12 = m[9] - sy * cz
    safe = n >= 2.0
    a00 = jnp.where(safe, a00, 1.0)
    a11 = jnp.where(safe, a11, 2.0)
    a22 = jnp.where(safe, a22, 3.0)
    a01 = jnp.where(safe, a01, 0.0)
    a02 = jnp.where(safe, a02, 0.0)
    a12 = jnp.where(safe, a12, 0.0)
    g00, g01, g02, g11, g12, g22 = a00, a01, a02, a11, a12, a22

    one = jnp.ones_like(a00)
    zero = jnp.zeros_like(a00)
    v00, v01, v02 = one, zero, zero
    v10, v11, v12 = zero, one, zero
    v20, v21, v22 = zero, zero, one

    for _ in range(3):
        c, s = _jacobi_rot(a00, a11, a01)
        a00, a11 = (c * c * a00 - 2 * s * c * a01 + s * s * a11,
                    s * s * a00 + 2 * s * c * a01 + c * c * a11)
        a02, a12 = c * a02 - s * a12, s * a02 + c * a12
        a01 = zero
        v00, v01 = c * v00 - s * v01, s * v00 + c * v01
        v10, v11 = c * v10 - s * v11, s * v10 + c * v11
        v20, v21 = c * v20 - s * v21, s * v20 + c * v21

        c, s = _jacobi_rot(a00, a22, a02)
        a00, a22 = (c * c * a00 - 2 * s * c * a02 + s * s * a22,
                    s * s * a00 + 2 * s * c * a02 + c * c * a22)
        a01, a12 = c * a01 - s * a12, s * a01 + c * a12
        a02 = zero
        v00, v02 = c * v00 - s * v02, s * v00 + c * v02
        v10, v12 = c * v10 - s * v12, s * v10 + c * v12
        v20, v22 = c * v20 - s * v22, s * v20 + c * v22

        c, s = _jacobi_rot(a11, a22, a12)
        a11, a22 = (c * c * a11 - 2 * s * c * a12 + s * s * a22,
                    s * s * a11 + 2 * s * c * a12 + c * c * a22)
        a01, a02 = c * a01 - s * a02, s * a01 + c * a02
        a12 = zero
        v01, v02 = c * v01 - s * v02, s * v01 + c * v02
        v11, v12 = c * v11 - s * v12, s * v11 + c * v12
        v21, v22 = c * v21 - s * v22, s * v21 + c * v22

    d0, d1, d2 = a00, a11, a22
    w2 = jnp.maximum(jnp.maximum(d0, d1), d2)
    w0 = jnp.minimum(jnp.minimum(d0, d1), d2)
    w1 = d0 + d1 + d2 - w2 - w0
    is0 = (d0 >= d1) & (d0 >= d2)
    is1 = jnp.logical_not(is0) & (d1 >= d2)
    v0x = jnp.where(is0, v00, jnp.where(is1, v01, v02))
    v0y = jnp.where(is0, v10, jnp.where(is1, v11, v12))
    v0z = jnp.where(is0, v20, jnp.where(is1, v21, v22))

    w2s = jnp.where(w2 != 0.0, w2, 1.0)
    dirwt = 1.0 - w1 / w2s

    feats = [
        jnp.where(safe, cx, sx),
        jnp.where(safe, cy, sy),
        jnp.where(safe, cz, sz),
        jnp.where(safe, g00 / w2s, 0.0),
        jnp.where(safe, g01 / w2s, 0.0),
        jnp.where(safe, g02 / w2s, 0.0),
        jnp.where(safe, g01 / w2s, 0.0),
        jnp.where(safe, g11 / w2s, 0.0),
        jnp.where(safe, g12 / w2s, 0.0),
        jnp.where(safe, g02 / w2s, 0.0),
        jnp.where(safe, g12 / w2s, 0.0),
        jnp.where(safe, g22 / w2s, 0.0),
        jnp.where(safe, dirwt * v0x, 0.0),
        jnp.where(safe, dirwt * v0y, 0.0),
        jnp.where(safe, dirwt * v0z, 0.0),
        n,
    ]
    for j in range(16):
        featref[j] = feats[j]
    pars = [cx, cy, cz, v0x, v0y, v0z]
    for j in range(6):
        parref[j] = pars[j]
    for j in range(6, 8):
        parref[j] = zero


def _tc1(momT):
    return pl.pallas_call(
        _tc1_body,
        grid=(GRID,),
        in_specs=[pl.BlockSpec((2, 10, 8, 128), lambda i: (0, 0, i, 0))],
        out_specs=[pl.BlockSpec((16, 8, 128), lambda i: (0, i, 0)),
                   pl.BlockSpec((8, 8, 128), lambda i: (0, i, 0))],
        out_shape=[jax.ShapeDtypeStruct((16, G, 128), jnp.float32),
                   jax.ShapeDtypeStruct((8, G, 128), jnp.float32)],
    )(momT)


def _tc2_body(featref, scref, outref):
    sc = scref[0] + scref[1]
    n = featref[15]
    flip = (n >= 2.0) & (sc < 0.0)
    fac = jnp.where(flip, -1.0, 1.0)
    for j in range(12):
        outref[j] = featref[j]
    for j in (12, 13, 14):
        outref[j] = featref[j] * fac
    outref[15] = n


def _tc2(feats0T, scs):
    return pl.pallas_call(
        _tc2_body,
        grid=(GRID,),
        in_specs=[pl.BlockSpec((16, 8, 128), lambda i: (0, i, 0)),
                  pl.BlockSpec((2, 8, 128), lambda i: (0, i, 0))],
        out_specs=pl.BlockSpec((16, 8, 128), lambda i: (0, i, 0)),
        out_shape=jax.ShapeDtypeStruct((16, G, 128), jnp.float32),
    )(feats0T, scs)


def kernel(data, segment_ids):
    seg2d = segment_ids.astype(jnp.int32).reshape(NB, 128)
    xs = data[:, 1].reshape(NB, 128)
    ys = data[:, 2].reshape(NB, 128)
    zs = data[:, 3].reshape(NB, 128)

    mom = _sc_moments(xs, ys, zs, seg2d)
    momT = mom.reshape(2, 10, G, 128)
    feats0T, paramsT = _tc1(momT)
    params = paramsT.reshape(8, CPAD)[:, :C].T

    sc_acc = _sc_orient(xs, ys, zs, seg2d, params)
    scs = sc_acc.reshape(2, G, 128)
    outT = _tc2(feats0T, scs)
    return outT.reshape(16, CPAD)[:, :C].T


# speedup vs baseline: 401.1279x; 1.1238x over previous
import functools

import jax
import jax.numpy as jnp
from jax import lax
from jax.experimental import pallas as pl
from jax.experimental.pallas import tpu as pltpu
from jax.experimental.pallas import tpu_sc as plsc

N = 1_600_000
C = 50_000

NB = N // 128
BPC = 20
PB = BPC * 128
NCHUNKS = N // PB
NW = 32
TMAX = (NCHUNKS + NW - 1) // NW

CPAD = 50_176
G = CPAD // 128
GRID = G // 8
RPT = CPAD // 16
ZR = RPT // 4

_MESH = plsc.VectorSubcoreMesh(core_axis_name="c", subcore_axis_name="s")
_SC_PARAMS = pltpu.CompilerParams(use_tc_tiling_on_sc=False,
                                  needs_layout_passes=False)


def _zero_rows(ref, nrows):
    zero16 = jnp.zeros((16,), jnp.float32)

    def body(i, carry):
        ref[i, :] = zero16
        return carry

    lax.fori_loop(0, nrows, body, 0)


def _zero_rows8(ref, nrows):
    zero16 = jnp.zeros((16,), jnp.float32)
    iota = lax.iota(jnp.int32, 16)

    def body(k, carry):
        flat = k * 16 + iota
        plsc.store_scatter(ref, [lax.shift_right_logical(flat, 3),
                                 lax.bitwise_and(flat, 7)], zero16)
        return carry

    lax.fori_loop(0, nrows // 2, body, 0)


def _sc_prologue(acc, rowbuf, zbuf, sid):
    _zero_rows(rowbuf, 128)
    _zero_rows(zbuf, ZR)
    base = sid * RPT
    for r in range(4):
        pltpu.sync_copy(zbuf, acc.at[pl.ds(base + r * ZR, ZR), :])


def _fire_inputs(x_hbm, y_hbm, z_hbm, seg_hbm, xbuf, ybuf, zbuf, sbuf,
                 slot, chunk, insem):
    blk0 = chunk * BPC
    pltpu.async_copy(x_hbm.at[pl.ds(blk0, BPC), :], xbuf.at[slot], insem)
    pltpu.async_copy(y_hbm.at[pl.ds(blk0, BPC), :], ybuf.at[slot], insem)
    pltpu.async_copy(z_hbm.at[pl.ds(blk0, BPC), :], zbuf.at[slot], insem)
    pltpu.async_copy(seg_hbm.at[pl.ds(blk0, BPC), :], sbuf.at[slot], insem)


def _drain_inputs(x_hbm, y_hbm, z_hbm, seg_hbm, xbuf, ybuf, zbuf, sbuf,
                  slot, insem):
    pltpu.make_async_copy(x_hbm.at[pl.ds(0, BPC), :], xbuf.at[slot], insem).wait()
    pltpu.make_async_copy(y_hbm.at[pl.ds(0, BPC), :], ybuf.at[slot], insem).wait()
    pltpu.make_async_copy(z_hbm.at[pl.ds(0, BPC), :], zbuf.at[slot], insem).wait()
    pltpu.make_async_copy(seg_hbm.at[pl.ds(0, BPC), :], sbuf.at[slot], insem).wait()


@functools.partial(
    pl.kernel,
    out_type=jax.ShapeDtypeStruct((2, 10, CPAD), jnp.float32),
    mesh=_MESH,
    scratch_types=[
        pltpu.VMEM_SHARED((CPAD, 16), jnp.float32),
        pltpu.VMEM((2, BPC, 128), jnp.float32),
        pltpu.VMEM((2, BPC, 128), jnp.float32),
        pltpu.VMEM((2, BPC, 128), jnp.float32),
        pltpu.VMEM((2, BPC, 128), jnp.int32),
        pltpu.VMEM((2, 128, 16), jnp.float32),
        pltpu.VMEM((ZR, 16), jnp.float32),
        pltpu.VMEM((10, ZR), jnp.float32),
        pltpu.SemaphoreType.DMA,
        pltpu.SemaphoreType.DMA,
        pltpu.SemaphoreType.DMA,
    ],
    compiler_params=_SC_PARAMS,
)
def _sc_moments(x_hbm, y_hbm, z_hbm, seg_hbm, mom_out,
                acc, xbuf, ybuf, zbuf, sbuf, rowbuf, zrow, trows,
                insem, scsem0, scsem1):
    cid = lax.axis_index("c")
    sid = lax.axis_index("s")
    wid = cid * 16 + sid
    _zero_rows(rowbuf.at[0], 128)
    _zero_rows(rowbuf.at[1], 128)
    _zero_rows(zrow, ZR)
    base0 = sid * RPT
    for r in range(4):
        pltpu.sync_copy(zrow, acc.at[pl.ds(base0 + r * ZR, ZR), :])
    plsc.subcore_barrier()
    iota = lax.iota(jnp.int32, 16)
    ones = jnp.full((16,), 1.0, jnp.float32)
    for r in range(2):
        for g in range(8):
            plsc.store_scatter(rowbuf.at[r],
                               [iota + g * 16, jnp.zeros((16,), jnp.int32)],
                               ones)
    scsems = (scsem0, scsem1)
    drain_dst = (rowbuf.at[0], rowbuf.at[1])

    _fire_inputs(x_hbm, y_hbm, z_hbm, seg_hbm, xbuf, ybuf, zbuf, sbuf,
                 0, wid, insem)

    def chunk_body(t, carry):
        chunk = wid + NW * t
        slot = lax.rem(t, 2)

        @pl.when(chunk < NCHUNKS)
        def _():
            _drain_inputs(x_hbm, y_hbm, z_hbm, seg_hbm, xbuf, ybuf, zbuf,
                          sbuf, slot, insem)
            nxt = chunk + NW

            @pl.when(nxt < NCHUNKS)
            def _():
                _fire_inputs(x_hbm, y_hbm, z_hbm, seg_hbm, xbuf, ybuf, zbuf,
                             sbuf, 1 - slot, nxt, insem)

            xb = xbuf.at[slot]
            yb = ybuf.at[slot]
            zb = zbuf.at[slot]
            sb = sbuf.at[slot]

            def pair_body(p, c2):
                for r in range(2):
                    b = 2 * p + r
                    rb = rowbuf.at[r]

                    @pl.when((p > 0) | (t > 0))
                    def _():
                        pltpu.make_async_copy(
                            x_hbm.at[pl.ds(0, 128), pl.ds(0, 16)], rb,
                            scsems[r]).wait()
                    for g in range(8):
                        sl = pl.ds(g * 16, 16)
                        rr = iota + g * 16
                        vx = xb[b, sl]
                        vy = yb[b, sl]
                        vz = zb[b, sl]

                        def put(col, val):
                            plsc.store_scatter(
                                rb, [rr, jnp.full((16,), col, jnp.int32)], val)

                        put(1, vx)
                        put(2, vy)
                        put(3, vz)
                        put(4, vx * vx)
                        put(5, vy * vy)
                        put(6, vz * vz)
                        put(7, vx * vy)
                        put(8, vx * vz)
                        put(9, vy * vz)
                    pltpu.async_copy(rb, acc.at[sb.at[b]], scsems[r], add=True)
                return c2

            lax.fori_loop(0, BPC // 2, pair_body, 0)

        return carry

    lax.fori_loop(0, TMAX, chunk_body, 0)
    for r in range(2):
        pltpu.make_async_copy(x_hbm.at[pl.ds(0, 128), pl.ds(0, 16)],
                              drain_dst[r], scsems[r]).wait()
    plsc.subcore_barrier()
    for ch in range(4):
        base = sid * RPT + ch * ZR
        pltpu.sync_copy(acc.at[pl.ds(base, ZR), :], zrow)

        def grp_body(g2, c3):
            rows = iota + g2 * 16
            for j in range(10):
                v = plsc.load_gather(zrow, [rows, jnp.full((16,), j, jnp.int32)])
                trows[j, pl.ds(g2 * 16, 16)] = v
            return c3

        lax.fori_loop(0, ZR // 16, grp_body, 0)
        pltpu.sync_copy(trows, mom_out.at[cid, :, pl.ds(base, ZR)])


@functools.partial(
    pl.kernel,
    out_type=jax.ShapeDtypeStruct((2, CPAD), jnp.float32),
    mesh=_MESH,
    scratch_types=[
        pltpu.VMEM_SHARED((CPAD, 8), jnp.float32),
        pltpu.VMEM_SHARED((CPAD, 8), jnp.float32),
        pltpu.VMEM((2, BPC, 128), jnp.float32),
        pltpu.VMEM((2, BPC, 128), jnp.float32),
        pltpu.VMEM((2, BPC, 128), jnp.float32),
        pltpu.VMEM((2, BPC, 128), jnp.int32),
        pltpu.VMEM((2, 128, 8), jnp.float32),
        pltpu.VMEM((ZR, 8), jnp.float32),
        pltpu.VMEM((2, 128, 8), jnp.float32),
        pltpu.VMEM((ZR,), jnp.float32),
        pltpu.VMEM((6, 13, 128), jnp.float32),
        pltpu.VMEM((1664, 8), jnp.float32),
        pltpu.SemaphoreType.DMA,
        pltpu.SemaphoreType.DMA,
        pltpu.SemaphoreType.DMA,
        pltpu.SemaphoreType.DMA,
        pltpu.SemaphoreType.DMA,
    ],
    compiler_params=_SC_PARAMS,
)
def _sc_orient(x_hbm, y_hbm, z_hbm, seg_hbm, params_hbm, sc_out,
               acc, ptab, xbuf, ybuf, zbuf, sbuf, rowbuf, zrow, prow, srow,
               pstage, pbuf,
               insem, scsem0, scsem1, gsem0, gsem1):
    cid = lax.axis_index("c")
    sid = lax.axis_index("s")
    wid = cid * 16 + sid
    iota = lax.iota(jnp.int32, 16)
    _zero_rows8(rowbuf.at[0], 128)
    _zero_rows8(rowbuf.at[1], 128)
    _zero_rows8(zrow, ZR)
    base0 = sid * RPT
    for r in range(4):
        pltpu.sync_copy(zrow, acc.at[pl.ds(base0 + r * ZR, ZR), :])

    def stage(gr0, nr):
        for j in range(6):
            pltpu.sync_copy(params_hbm.at[j, pl.ds(gr0, nr), :],
                            pstage.at[j, pl.ds(0, nr), :])

        def gg_body(gg, c0):
            row = lax.div(gg, jnp.int32(8))
            off = lax.rem(gg, jnp.int32(8)) * 16
            rr = iota + gg * 16
            for j in range(6):
                v = pstage[j, row, pl.ds(off, 16)]
                plsc.store_scatter(pbuf, [rr, jnp.full((16,), j, jnp.int32)], v)
            return c0

        lax.fori_loop(0, nr * 8, gg_body, 0)
        pltpu.sync_copy(pbuf.at[pl.ds(0, nr * 128), :],
                        ptab.at[pl.ds(gr0 * 128, nr * 128), :])

    @pl.when(sid < 8)
    def _():
        stage(sid * 25, 13)
        stage(sid * 25 + 13, 12)

    @pl.when(sid >= 8)
    def _():
        stage(200 + (sid - 8) * 24, 12)
        stage(200 + (sid - 8) * 24 + 12, 12)

    plsc.subcore_barrier()
    col0 = jnp.zeros((16,), jnp.int32)
    magic = jnp.full((16,), 0x5F3759DF, jnp.int32)
    one_i = jnp.full((16,), 1, jnp.int32)
    scsems = (scsem0, scsem1)
    gsems = (gsem0, gsem1)

    _fire_inputs(x_hbm, y_hbm, z_hbm, seg_hbm, xbuf, ybuf, zbuf, sbuf,
                 0, wid, insem)

    def chunk_body(t, carry):
        chunk = wid + NW * t
        slot = lax.rem(t, 2)

        @pl.when(chunk < NCHUNKS)
        def _():
            _drain_inputs(x_hbm, y_hbm, z_hbm, seg_hbm, xbuf, ybuf, zbuf,
                          sbuf, slot, insem)
            nxt = chunk + NW

            @pl.when(nxt < NCHUNKS)
            def _():
                _fire_inputs(x_hbm, y_hbm, z_hbm, seg_hbm, xbuf, ybuf, zbuf,
                             sbuf, 1 - slot, nxt, insem)

            xb = xbuf.at[slot]
            yb = ybuf.at[slot]
            zb = zbuf.at[slot]
            sb = sbuf.at[slot]
            pltpu.async_copy(ptab.at[sb.at[0]], prow.at[0], gsem0)

            def pair_body(p, c2):
                for r in range(2):
                    b = 2 * p + r
                    rb = rowbuf.at[r]
                    pb = prow.at[r]
                    pltpu.make_async_copy(
                        x_hbm.at[pl.ds(0, 128), pl.ds(0, 8)], pb,
                        gsems[r]).wait()

                    @pl.when(b + 1 < BPC)
                    def _():
                        pltpu.async_copy(ptab.at[sb.at[b + 1]],
                                         prow.at[1 - r], gsems[1 - r])

                    @pl.when((p > 0) | (t > 0))
                    def _():
                        pltpu.make_async_copy(
                            x_hbm.at[pl.ds(0, 128), pl.ds(0, 8)], rb,
                            scsems[r]).wait()
                    for g in range(8):
                        sl = pl.ds(g * 16, 16)
                        rr = iota + g * 16
                        x = xb[b, sl]
                        y = yb[b, sl]
                        z = zb[b, sl]
                        cx = plsc.load_gather(pb, [rr, jnp.full((16,), 0, jnp.int32)])
                        cy = plsc.load_gather(pb, [rr, jnp.full((16,), 1, jnp.int32)])
                        cz = plsc.load_gather(pb, [rr, jnp.full((16,), 2, jnp.int32)])
                        vx = plsc.load_gather(pb, [rr, jnp.full((16,), 3, jnp.int32)])
                        vy = plsc.load_gather(pb, [rr, jnp.full((16,), 4, jnp.int32)])
                        vz = plsc.load_gather(pb, [rr, jnp.full((16,), 5, jnp.int32)])
                        xcx = x - cx
                        xcy = y - cy
                        xcz = z - cz
                        x0 = xcx * vx + xcy * vy + xcz * vz
                        d = xcx * xcx + xcy * xcy + xcz * xcz - x0 * x0
                        d = jnp.maximum(d, 0.0)
                        rv = plsc.bitcast(magic - lax.shift_right_logical(
                            plsc.bitcast(d, jnp.int32), one_i), jnp.float32)
                        for _ in range(3):
                            h = 0.5 * d * rv
                            rv = rv * (1.5 - h * rv)
                        t_val = x0 * (d * rv)
                        plsc.store_scatter(rb, [rr, col0], t_val)
                    pltpu.async_copy(rb, acc.at[sb.at[b]], scsems[r], add=True)
                return c2

            lax.fori_loop(0, BPC // 2, pair_body, 0)

        return carry

    lax.fori_loop(0, TMAX, chunk_body, 0)
    for r in range(2):
        pltpu.make_async_copy(x_hbm.at[pl.ds(0, 128), pl.ds(0, 8)],
                              rowbuf.at[r], scsems[r]).wait()
    plsc.subcore_barrier()
    for ch in range(4):
        base = sid * RPT + ch * ZR
        pltpu.sync_copy(acc.at[pl.ds(base, ZR), :], zrow)

        def grp_body(g2, c3):
            v = plsc.load_gather(zrow, [iota + g2 * 16, col0])
            srow[pl.ds(g2 * 16, 16)] = v
            return c3

        lax.fori_loop(0, ZR // 16, grp_body, 0)
        pltpu.sync_copy(srow, sc_out.at[cid, pl.ds(base, ZR)])


def _jacobi_rot(app, aqq, apq):
    small = jnp.abs(apq) <= 1e-30
    apq_s = jnp.where(small, 1.0, apq)
    tau = (aqq - app) / (2.0 * apq_s)
    t = jnp.sign(tau) / (jnp.abs(tau) + jnp.sqrt(1.0 + tau * tau))
    t = jnp.where(tau == 0.0, 1.0, t)
    c = 1.0 / jnp.sqrt(1.0 + t * t)
    s = t * c
    c = jnp.where(small, 1.0, c)
    s = jnp.where(small, 0.0, s)
    return c, s


def _tc1_body(momref, featref, parref):
    m = [momref[0, j] + momref[1, j] for j in range(10)]
    n = m[0]
    n_safe = jnp.maximum(n, 1.0)
    sx, sy, sz = m[1], m[2], m[3]
    cx, cy, cz = sx / n_safe, sy / n_safe, sz / n_safe
    a00 = m[4] - sx * cx
    a11 = m[5] - sy * cy
    a22 = m[6] - sz * cz
    a01 = m[7] - sx * cy
    a02 = m[8] - sx * cz
    a12 = m[9] - sy * cz
    safe = n >= 2.0
    a00 = jnp.where(safe, a00, 1.0)
    a11 = jnp.where(safe, a11, 2.0)
    a22 = jnp.where(safe, a22, 3.0)
    a01 = jnp.where(safe, a01, 0.0)
    a02 = jnp.where(safe, a02, 0.0)
    a12 = jnp.where(safe, a12, 0.0)
    g00, g01, g02, g11, g12, g22 = a00, a01, a02, a11, a12, a22

    one = jnp.ones_like(a00)
    zero = jnp.zeros_like(a00)
    v00, v01, v02 = one, zero, zero
    v10, v11, v12 = zero, one, zero
    v20, v21, v22 = zero, zero, one

    for _ in range(3):
        c, s = _jacobi_rot(a00, a11, a01)
        a00, a11 = (c * c * a00 - 2 * s * c * a01 + s * s * a11,
                    s * s * a00 + 2 * s * c * a01 + c * c * a11)
        a02, a12 = c * a02 - s * a12, s * a02 + c * a12
        a01 = zero
        v00, v01 = c * v00 - s * v01, s * v00 + c * v01
        v10, v11 = c * v10 - s * v11, s * v10 + c * v11
        v20, v21 = c * v20 - s * v21, s * v20 + c * v21

        c, s = _jacobi_rot(a00, a22, a02)
        a00, a22 = (c * c * a00 - 2 * s * c * a02 + s * s * a22,
                    s * s * a00 + 2 * s * c * a02 + c * c * a22)
        a01, a12 = c * a01 - s * a12, s * a01 + c * a12
        a02 = zero
        v00, v02 = c * v00 - s * v02, s * v00 + c * v02
        v10, v12 = c * v10 - s * v12, s * v10 + c * v12
        v20, v22 = c * v20 - s * v22, s * v20 + c * v22

        c, s = _jacobi_rot(a11, a22, a12)
        a11, a22 = (c * c * a11 - 2 * s * c * a12 + s * s * a22,
                    s * s * a11 + 2 * s * c * a12 + c * c * a22)
        a01, a02 = c * a01 - s * a02, s * a01 + c * a02
        a12 = zero
        v01, v02 = c * v01 - s * v02, s * v01 + c * v02
        v11, v12 = c * v11 - s * v12, s * v11 + c * v12
        v21, v22 = c * v21 - s * v22, s * v21 + c * v22

    d0, d1, d2 = a00, a11, a22
    w2 = jnp.maximum(jnp.maximum(d0, d1), d2)
    w0 = jnp.minimum(jnp.minimum(d0, d1), d2)
    w1 = d0 + d1 + d2 - w2 - w0
    is0 = (d0 >= d1) & (d0 >= d2)
    is1 = jnp.logical_not(is0) & (d1 >= d2)
    v0x = jnp.where(is0, v00, jnp.where(is1, v01, v02))
    v0y = jnp.where(is0, v10, jnp.where(is1, v11, v12))
    v0z = jnp.where(is0, v20, jnp.where(is1, v21, v22))

    w2s = jnp.where(w2 != 0.0, w2, 1.0)
    dirwt = 1.0 - w1 / w2s

    feats = [
        jnp.where(safe, cx, sx),
        jnp.where(safe, cy, sy),
        jnp.where(safe, cz, sz),
        jnp.where(safe, g00 / w2s, 0.0),
        jnp.where(safe, g01 / w2s, 0.0),
        jnp.where(safe, g02 / w2s, 0.0),
        jnp.where(safe, g01 / w2s, 0.0),
        jnp.where(safe, g11 / w2s, 0.0),
        jnp.where(safe, g12 / w2s, 0.0),
        jnp.where(safe, g02 / w2s, 0.0),
        jnp.where(safe, g12 / w2s, 0.0),
        jnp.where(safe, g22 / w2s, 0.0),
        jnp.where(safe, dirwt * v0x, 0.0),
        jnp.where(safe, dirwt * v0y, 0.0),
        jnp.where(safe, dirwt * v0z, 0.0),
        n,
    ]
    for j in range(16):
        featref[j] = feats[j]
    pars = [cx, cy, cz, v0x, v0y, v0z]
    for j in range(6):
        parref[j] = pars[j]


def _tc1(momT):
    return pl.pallas_call(
        _tc1_body,
        grid=(GRID,),
        in_specs=[pl.BlockSpec((2, 10, 8, 128), lambda i: (0, 0, i, 0))],
        out_specs=[pl.BlockSpec((16, 8, 128), lambda i: (0, i, 0)),
                   pl.BlockSpec((6, 8, 128), lambda i: (0, i, 0))],
        out_shape=[jax.ShapeDtypeStruct((16, G, 128), jnp.float32),
                   jax.ShapeDtypeStruct((6, G, 128), jnp.float32)],
    )(momT)


def _tc2_body(featref, scref, outref):
    sc = scref[0] + scref[1]
    n = featref[3]
    flip = (n >= 2.0) & (sc < 0.0)
    fac = jnp.where(flip, -1.0, 1.0)
    outref[0] = featref[0] * fac
    outref[1] = featref[1] * fac
    outref[2] = featref[2] * fac
    outref[3] = n


def _tc2(feats0T, scs):
    return pl.pallas_call(
        _tc2_body,
        grid=(GRID,),
        in_specs=[pl.BlockSpec((4, 8, 128), lambda i: (3, i, 0)),
                  pl.BlockSpec((2, 8, 128), lambda i: (0, i, 0))],
        out_specs=pl.BlockSpec((4, 8, 128), lambda i: (3, i, 0)),
        out_shape=jax.ShapeDtypeStruct((16, G, 128), jnp.float32),
        input_output_aliases={0: 0},
    )(feats0T, scs)


def kernel(data, segment_ids):
    seg2d = segment_ids.astype(jnp.int32).reshape(NB, 128)
    xs = data[:, 1].reshape(NB, 128)
    ys = data[:, 2].reshape(NB, 128)
    zs = data[:, 3].reshape(NB, 128)

    mom = _sc_moments(xs, ys, zs, seg2d)
    momT = mom.reshape(2, 10, G, 128)
    feats0T, params = _tc1(momT)

    sc_acc = _sc_orient(xs, ys, zs, seg2d, params)
    scs = sc_acc.reshape(2, G, 128)
    outT = _tc2(feats0T, scs)
    return outT.reshape(16, CPAD)[:, :C].T


# speedup vs baseline: 457.4275x; 1.1404x over previous
import functools

import jax
import jax.numpy as jnp
from jax import lax
from jax.experimental import pallas as pl
from jax.experimental.pallas import tpu as pltpu
from jax.experimental.pallas import tpu_sc as plsc

N = 1_600_000
C = 50_000

NB = N // 128
BPC = 20
PB = BPC * 128
NCHUNKS = N // PB
NW = 32
TMAX = (NCHUNKS + NW - 1) // NW

CPAD = 50_176
G = CPAD // 128
GRID = G // 8
RPT = CPAD // 16
ZR = RPT // 4

_MESH = plsc.VectorSubcoreMesh(core_axis_name="c", subcore_axis_name="s")
_SC_PARAMS = pltpu.CompilerParams(use_tc_tiling_on_sc=False,
                                  needs_layout_passes=False)


def _zero_rows(ref, nrows):
    zero16 = jnp.zeros((16,), jnp.float32)

    def body(i, carry):
        ref[i, :] = zero16
        return carry

    lax.fori_loop(0, nrows, body, 0)


def _zero_rows8(ref, nrows):
    zero16 = jnp.zeros((16,), jnp.float32)
    iota = lax.iota(jnp.int32, 16)

    def body(k, carry):
        flat = k * 16 + iota
        plsc.store_scatter(ref, [lax.shift_right_logical(flat, 3),
                                 lax.bitwise_and(flat, 7)], zero16)
        return carry

    lax.fori_loop(0, nrows // 2, body, 0)


def _sc_prologue(acc, rowbuf, zbuf, sid):
    _zero_rows(rowbuf, 128)
    _zero_rows(zbuf, ZR)
    base = sid * RPT
    for r in range(4):
        pltpu.sync_copy(zbuf, acc.at[pl.ds(base + r * ZR, ZR), :])


def _fire_inputs(d3_hbm, seg_hbm, xbuf, ybuf, zbuf, sbuf,
                 slot, chunk, insem):
    blk0 = chunk * BPC
    pltpu.async_copy(d3_hbm.at[pl.ds(blk0, BPC), 1, :], xbuf.at[slot], insem)
    pltpu.async_copy(d3_hbm.at[pl.ds(blk0, BPC), 2, :], ybuf.at[slot], insem)
    pltpu.async_copy(d3_hbm.at[pl.ds(blk0, BPC), 3, :], zbuf.at[slot], insem)
    pltpu.async_copy(seg_hbm.at[pl.ds(blk0, BPC), :], sbuf.at[slot], insem)


def _drain_inputs(d3_hbm, seg_hbm, xbuf, ybuf, zbuf, sbuf,
                  slot, insem):
    pltpu.make_async_copy(d3_hbm.at[pl.ds(0, BPC), 1, :], xbuf.at[slot], insem).wait()
    pltpu.make_async_copy(d3_hbm.at[pl.ds(0, BPC), 2, :], ybuf.at[slot], insem).wait()
    pltpu.make_async_copy(d3_hbm.at[pl.ds(0, BPC), 3, :], zbuf.at[slot], insem).wait()
    pltpu.make_async_copy(seg_hbm.at[pl.ds(0, BPC), :], sbuf.at[slot], insem).wait()


@functools.partial(
    pl.kernel,
    out_type=jax.ShapeDtypeStruct((2, 10, CPAD), jnp.float32),
    mesh=_MESH,
    scratch_types=[
        pltpu.VMEM_SHARED((CPAD, 16), jnp.float32),
        pltpu.VMEM((2, BPC, 128), jnp.float32),
        pltpu.VMEM((2, BPC, 128), jnp.float32),
        pltpu.VMEM((2, BPC, 128), jnp.float32),
        pltpu.VMEM((2, BPC, 128), jnp.int32),
        pltpu.VMEM((2, 128, 16), jnp.float32),
        pltpu.VMEM((ZR, 16), jnp.float32),
        pltpu.VMEM((10, ZR), jnp.float32),
        pltpu.SemaphoreType.DMA,
        pltpu.SemaphoreType.DMA,
        pltpu.SemaphoreType.DMA,
    ],
    compiler_params=_SC_PARAMS,
)
def _sc_moments(d3_hbm, seg_hbm, mom_out,
                acc, xbuf, ybuf, zbuf, sbuf, rowbuf, zrow, trows,
                insem, scsem0, scsem1):
    cid = lax.axis_index("c")
    sid = lax.axis_index("s")
    wid = cid * 16 + sid
    _zero_rows(rowbuf.at[0], 128)
    _zero_rows(rowbuf.at[1], 128)
    _zero_rows(zrow, ZR)
    base0 = sid * RPT
    for r in range(4):
        pltpu.sync_copy(zrow, acc.at[pl.ds(base0 + r * ZR, ZR), :])
    plsc.subcore_barrier()
    iota = lax.iota(jnp.int32, 16)
    ones = jnp.full((16,), 1.0, jnp.float32)
    for r in range(2):
        for g in range(8):
            plsc.store_scatter(rowbuf.at[r],
                               [iota + g * 16, jnp.zeros((16,), jnp.int32)],
                               ones)
    scsems = (scsem0, scsem1)
    drain_dst = (rowbuf.at[0], rowbuf.at[1])

    _fire_inputs(d3_hbm, seg_hbm, xbuf, ybuf, zbuf, sbuf,
                 0, wid, insem)

    def chunk_body(t, carry):
        chunk = wid + NW * t
        slot = lax.rem(t, 2)

        @pl.when(chunk < NCHUNKS)
        def _():
            _drain_inputs(d3_hbm, seg_hbm, xbuf, ybuf, zbuf,
                          sbuf, slot, insem)
            nxt = chunk + NW

            @pl.when(nxt < NCHUNKS)
            def _():
                _fire_inputs(d3_hbm, seg_hbm, xbuf, ybuf, zbuf,
                             sbuf, 1 - slot, nxt, insem)

            xb = xbuf.at[slot]
            yb = ybuf.at[slot]
            zb = zbuf.at[slot]
            sb = sbuf.at[slot]

            def pair_body(p, c2):
                for r in range(2):
                    b = 2 * p + r
                    rb = rowbuf.at[r]

                    @pl.when((p > 0) | (t > 0))
                    def _():
                        pltpu.make_async_copy(
                            d3_hbm.at[pl.ds(0, 128), 0, pl.ds(0, 16)], rb,
                            scsems[r]).wait()
                    for g in range(8):
                        sl = pl.ds(g * 16, 16)
                        rr = iota + g * 16
                        vx = xb[b, sl]
                        vy = yb[b, sl]
                        vz = zb[b, sl]

                        def put(col, val):
                            plsc.store_scatter(
                                rb, [rr, jnp.full((16,), col, jnp.int32)], val)

                        put(1, vx)
                        put(2, vy)
                        put(3, vz)
                        put(4, vx * vx)
                        put(5, vy * vy)
                        put(6, vz * vz)
                        put(7, vx * vy)
                        put(8, vx * vz)
                        put(9, vy * vz)
                    pltpu.async_copy(rb, acc.at[sb.at[b]], scsems[r], add=True)
                return c2

            lax.fori_loop(0, BPC // 2, pair_body, 0)

        return carry

    lax.fori_loop(0, TMAX, chunk_body, 0)
    for r in range(2):
        pltpu.make_async_copy(d3_hbm.at[pl.ds(0, 128), 0, pl.ds(0, 16)],
                              drain_dst[r], scsems[r]).wait()
    plsc.subcore_barrier()
    for ch in range(4):
        base = sid * RPT + ch * ZR
        pltpu.sync_copy(acc.at[pl.ds(base, ZR), :], zrow)

        def grp_body(g2, c3):
            rows = iota + g2 * 16
            for j in range(10):
                v = plsc.load_gather(zrow, [rows, jnp.full((16,), j, jnp.int32)])
                trows[j, pl.ds(g2 * 16, 16)] = v
            return c3

        lax.fori_loop(0, ZR // 16, grp_body, 0)
        pltpu.sync_copy(trows, mom_out.at[cid, :, pl.ds(base, ZR)])


@functools.partial(
    pl.kernel,
    out_type=jax.ShapeDtypeStruct((2, CPAD), jnp.float32),
    mesh=_MESH,
    scratch_types=[
        pltpu.VMEM_SHARED((CPAD, 8), jnp.float32),
        pltpu.VMEM_SHARED((CPAD, 8), jnp.float32),
        pltpu.VMEM((2, BPC, 128), jnp.float32),
        pltpu.VMEM((2, BPC, 128), jnp.float32),
        pltpu.VMEM((2, BPC, 128), jnp.float32),
        pltpu.VMEM((2, BPC, 128), jnp.int32),
        pltpu.VMEM((2, 128, 8), jnp.float32),
        pltpu.VMEM((ZR, 8), jnp.float32),
        pltpu.VMEM((2, 128, 8), jnp.float32),
        pltpu.VMEM((ZR,), jnp.float32),
        pltpu.VMEM((6, 13, 128), jnp.float32),
        pltpu.VMEM((1664, 8), jnp.float32),
        pltpu.SemaphoreType.DMA,
        pltpu.SemaphoreType.DMA,
        pltpu.SemaphoreType.DMA,
        pltpu.SemaphoreType.DMA,
        pltpu.SemaphoreType.DMA,
    ],
    compiler_params=_SC_PARAMS,
)
def _sc_orient(d3_hbm, seg_hbm, params_hbm, sc_out,
               acc, ptab, xbuf, ybuf, zbuf, sbuf, rowbuf, zrow, prow, srow,
               pstage, pbuf,
               insem, scsem0, scsem1, gsem0, gsem1):
    cid = lax.axis_index("c")
    sid = lax.axis_index("s")
    wid = cid * 16 + sid
    iota = lax.iota(jnp.int32, 16)
    _zero_rows8(rowbuf.at[0], 128)
    _zero_rows8(rowbuf.at[1], 128)
    _zero_rows8(zrow, ZR)
    base0 = sid * RPT
    for r in range(4):
        pltpu.sync_copy(zrow, acc.at[pl.ds(base0 + r * ZR, ZR), :])

    def stage(gr0, nr):
        for j in range(6):
            pltpu.sync_copy(params_hbm.at[j, pl.ds(gr0, nr), :],
                            pstage.at[j, pl.ds(0, nr), :])

        def gg_body(gg, c0):
            row = lax.div(gg, jnp.int32(8))
            off = lax.rem(gg, jnp.int32(8)) * 16
            rr = iota + gg * 16
            for j in range(6):
                v = pstage[j, row, pl.ds(off, 16)]
                plsc.store_scatter(pbuf, [rr, jnp.full((16,), j, jnp.int32)], v)
            return c0

        lax.fori_loop(0, nr * 8, gg_body, 0)
        pltpu.sync_copy(pbuf.at[pl.ds(0, nr * 128), :],
                        ptab.at[pl.ds(gr0 * 128, nr * 128), :])

    @pl.when(sid < 8)
    def _():
        stage(sid * 25, 13)
        stage(sid * 25 + 13, 12)

    @pl.when(sid >= 8)
    def _():
        stage(200 + (sid - 8) * 24, 12)
        stage(200 + (sid - 8) * 24 + 12, 12)

    plsc.subcore_barrier()
    col0 = jnp.zeros((16,), jnp.int32)
    magic = jnp.full((16,), 0x5F3759DF, jnp.int32)
    one_i = jnp.full((16,), 1, jnp.int32)
    scsems = (scsem0, scsem1)
    gsems = (gsem0, gsem1)

    _fire_inputs(d3_hbm, seg_hbm, xbuf, ybuf, zbuf, sbuf,
                 0, wid, insem)

    def chunk_body(t, carry):
        chunk = wid + NW * t
        slot = lax.rem(t, 2)

        @pl.when(chunk < NCHUNKS)
        def _():
            _drain_inputs(d3_hbm, seg_hbm, xbuf, ybuf, zbuf,
                          sbuf, slot, insem)
            nxt = chunk + NW

            @pl.when(nxt < NCHUNKS)
            def _():
                _fire_inputs(d3_hbm, seg_hbm, xbuf, ybuf, zbuf,
                             sbuf, 1 - slot, nxt, insem)

            xb = xbuf.at[slot]
            yb = ybuf.at[slot]
            zb = zbuf.at[slot]
            sb = sbuf.at[slot]
            pltpu.async_copy(ptab.at[sb.at[0]], prow.at[0], gsem0)

            def pair_body(p, c2):
                for r in range(2):
                    b = 2 * p + r
                    rb = rowbuf.at[r]
                    pb = prow.at[r]
                    pltpu.make_async_copy(
                        d3_hbm.at[pl.ds(0, 128), 0, pl.ds(0, 8)], pb,
                        gsems[r]).wait()

                    @pl.when(b + 1 < BPC)
                    def _():
                        pltpu.async_copy(ptab.at[sb.at[b + 1]],
                                         prow.at[1 - r], gsems[1 - r])

                    @pl.when((p > 0) | (t > 0))
                    def _():
                        pltpu.make_async_copy(
                            d3_hbm.at[pl.ds(0, 128), 0, pl.ds(0, 8)], rb,
                            scsems[r]).wait()
                    for g in range(8):
                        sl = pl.ds(g * 16, 16)
                        rr = iota + g * 16
                        x = xb[b, sl]
                        y = yb[b, sl]
                        z = zb[b, sl]
                        cx = plsc.load_gather(pb, [rr, jnp.full((16,), 0, jnp.int32)])
                        cy = plsc.load_gather(pb, [rr, jnp.full((16,), 1, jnp.int32)])
                        cz = plsc.load_gather(pb, [rr, jnp.full((16,), 2, jnp.int32)])
                        vx = plsc.load_gather(pb, [rr, jnp.full((16,), 3, jnp.int32)])
                        vy = plsc.load_gather(pb, [rr, jnp.full((16,), 4, jnp.int32)])
                        vz = plsc.load_gather(pb, [rr, jnp.full((16,), 5, jnp.int32)])
                        xcx = x - cx
                        xcy = y - cy
                        xcz = z - cz
                        x0 = xcx * vx + xcy * vy + xcz * vz
                        d = xcx * xcx + xcy * xcy + xcz * xcz - x0 * x0
                        d = jnp.maximum(d, 0.0)
                        rv = plsc.bitcast(magic - lax.shift_right_logical(
                            plsc.bitcast(d, jnp.int32), one_i), jnp.float32)
                        for _ in range(3):
                            h = 0.5 * d * rv
                            rv = rv * (1.5 - h * rv)
                        t_val = x0 * (d * rv)
                        plsc.store_scatter(rb, [rr, col0], t_val)
                    pltpu.async_copy(rb, acc.at[sb.at[b]], scsems[r], add=True)
                return c2

            lax.fori_loop(0, BPC // 2, pair_body, 0)

        return carry

    lax.fori_loop(0, TMAX, chunk_body, 0)
    for r in range(2):
        pltpu.make_async_copy(d3_hbm.at[pl.ds(0, 128), 0, pl.ds(0, 8)],
                              rowbuf.at[r], scsems[r]).wait()
    plsc.subcore_barrier()
    for ch in range(4):
        base = sid * RPT + ch * ZR
        pltpu.sync_copy(acc.at[pl.ds(base, ZR), :], zrow)

        def grp_body(g2, c3):
            v = plsc.load_gather(zrow, [iota + g2 * 16, col0])
            srow[pl.ds(g2 * 16, 16)] = v
            return c3

        lax.fori_loop(0, ZR // 16, grp_body, 0)
        pltpu.sync_copy(srow, sc_out.at[cid, pl.ds(base, ZR)])


def _jacobi_rot(app, aqq, apq):
    small = jnp.abs(apq) <= 1e-30
    apq_s = jnp.where(small, 1.0, apq)
    tau = (aqq - app) / (2.0 * apq_s)
    t = jnp.sign(tau) / (jnp.abs(tau) + jnp.sqrt(1.0 + tau * tau))
    t = jnp.where(tau == 0.0, 1.0, t)
    c = 1.0 / jnp.sqrt(1.0 + t * t)
    s = t * c
    c = jnp.where(small, 1.0, c)
    s = jnp.where(small, 0.0, s)
    return c, s


def _tc1_body(momref, featref, parref):
    m = [momref[0, j] + momref[1, j] for j in range(10)]
    n = m[0]
    n_safe = jnp.maximum(n, 1.0)
    sx, sy, sz = m[1], m[2], m[3]
    cx, cy, cz = sx / n_safe, sy / n_safe, sz / n_safe
    a00 = m[4] - sx * cx
    a11 = m[5] - sy * cy
    a22 = m[6] - sz * cz
    a01 = m[7] - sx * cy
    a02 = m[8] - sx * cz
    a12 = m[9] - sy * cz
    safe = n >= 2.0
    a00 = jnp.where(safe, a00, 1.0)
    a11 = jnp.where(safe, a11, 2.0)
    a22 = jnp.where(safe, a22, 3.0)
    a01 = jnp.where(safe, a01, 0.0)
    a02 = jnp.where(safe, a02, 0.0)
    a12 = jnp.where(safe, a12, 0.0)
    g00, g01, g02, g11, g12, g22 = a00, a01, a02, a11, a12, a22

    one = jnp.ones_like(a00)
    zero = jnp.zeros_like(a00)
    v00, v01, v02 = one, zero, zero
    v10, v11, v12 = zero, one, zero
    v20, v21, v22 = zero, zero, one

    for _ in range(3):
        c, s = _jacobi_rot(a00, a11, a01)
        a00, a11 = (c * c * a00 - 2 * s * c * a01 + s * s * a11,
                    s * s * a00 + 2 * s * c * a01 + c * c * a11)
        a02, a12 = c * a02 - s * a12, s * a02 + c * a12
        a01 = zero
        v00, v01 = c * v00 - s * v01, s * v00 + c * v01
        v10, v11 = c * v10 - s * v11, s * v10 + c * v11
        v20, v21 = c * v20 - s * v21, s * v20 + c * v21

        c, s = _jacobi_rot(a00, a22, a02)
        a00, a22 = (c * c * a00 - 2 * s * c * a02 + s * s * a22,
                    s * s * a00 + 2 * s * c * a02 + c * c * a22)
        a01, a12 = c * a01 - s * a12, s * a01 + c * a12
        a02 = zero
        v00, v02 = c * v00 - s * v02, s * v00 + c * v02
        v10, v12 = c * v10 - s * v12, s * v10 + c * v12
        v20, v22 = c * v20 - s * v22, s * v20 + c * v22

        c, s = _jacobi_rot(a11, a22, a12)
        a11, a22 = (c * c * a11 - 2 * s * c * a12 + s * s * a22,
                    s * s * a11 + 2 * s * c * a12 + c * c * a22)
        a01, a02 = c * a01 - s * a02, s * a01 + c * a02
        a12 = zero
        v01, v02 = c * v01 - s * v02, s * v01 + c * v02
        v11, v12 = c * v11 - s * v12, s * v11 + c * v12
        v21, v22 = c * v21 - s * v22, s * v21 + c * v22

    d0, d1, d2 = a00, a11, a22
    w2 = jnp.maximum(jnp.maximum(d0, d1), d2)
    w0 = jnp.minimum(jnp.minimum(d0, d1), d2)
    w1 = d0 + d1 + d2 - w2 - w0
    is0 = (d0 >= d1) & (d0 >= d2)
    is1 = jnp.logical_not(is0) & (d1 >= d2)
    v0x = jnp.where(is0, v00, jnp.where(is1, v01, v02))
    v0y = jnp.where(is0, v10, jnp.where(is1, v11, v12))
    v0z = jnp.where(is0, v20, jnp.where(is1, v21, v22))

    w2s = jnp.where(w2 != 0.0, w2, 1.0)
    dirwt = 1.0 - w1 / w2s

    feats = [
        jnp.where(safe, cx, sx),
        jnp.where(safe, cy, sy),
        jnp.where(safe, cz, sz),
        jnp.where(safe, g00 / w2s, 0.0),
        jnp.where(safe, g01 / w2s, 0.0),
        jnp.where(safe, g02 / w2s, 0.0),
        jnp.where(safe, g01 / w2s, 0.0),
        jnp.where(safe, g11 / w2s, 0.0),
        jnp.where(safe, g12 / w2s, 0.0),
        jnp.where(safe, g02 / w2s, 0.0),
        jnp.where(safe, g12 / w2s, 0.0),
        jnp.where(safe, g22 / w2s, 0.0),
        jnp.where(safe, dirwt * v0x, 0.0),
        jnp.where(safe, dirwt * v0y, 0.0),
        jnp.where(safe, dirwt * v0z, 0.0),
        n,
    ]
    for j in range(16):
        featref[j] = feats[j]
    pars = [cx, cy, cz, v0x, v0y, v0z]
    for j in range(6):
        parref[j] = pars[j]


def _tc1(momT):
    return pl.pallas_call(
        _tc1_body,
        grid=(GRID,),
        in_specs=[pl.BlockSpec((2, 10, 8, 128), lambda i: (0, 0, i, 0))],
        out_specs=[pl.BlockSpec((16, 8, 128), lambda i: (0, i, 0)),
                   pl.BlockSpec((6, 8, 128), lambda i: (0, i, 0))],
        out_shape=[jax.ShapeDtypeStruct((16, G, 128), jnp.float32),
                   jax.ShapeDtypeStruct((6, G, 128), jnp.float32)],
    )(momT)


def _tc2_body(featref, scref, outref):
    sc = scref[0] + scref[1]
    n = featref[3]
    flip = (n >= 2.0) & (sc < 0.0)
    fac = jnp.where(flip, -1.0, 1.0)
    outref[0] = featref[0] * fac
    outref[1] = featref[1] * fac
    outref[2] = featref[2] * fac
    outref[3] = n


def _tc2(feats0T, scs):
    return pl.pallas_call(
        _tc2_body,
        grid=(GRID,),
        in_specs=[pl.BlockSpec((4, 8, 128), lambda i: (3, i, 0)),
                  pl.BlockSpec((2, 8, 128), lambda i: (0, i, 0))],
        out_specs=pl.BlockSpec((4, 8, 128), lambda i: (3, i, 0)),
        out_shape=jax.ShapeDtypeStruct((16, G, 128), jnp.float32),
        input_output_aliases={0: 0},
    )(feats0T, scs)


def kernel(data, segment_ids):
    seg2d = segment_ids.astype(jnp.int32).reshape(NB, 128)
    d3 = jnp.pad(data, ((0, 0), (0, 3))).reshape(NB, 128, 8).transpose(0, 2, 1)

    mom = _sc_moments(d3, seg2d)
    momT = mom.reshape(2, 10, G, 128)
    feats0T, params = _tc1(momT)

    sc_acc = _sc_orient(d3, seg2d, params)
    scs = sc_acc.reshape(2, G, 128)
    outT = _tc2(feats0T, scs)
    return outT.reshape(16, CPAD)[:, :C].T


# speedup vs baseline: 524.1148x; 1.1458x over previous
import functools

import jax
import jax.numpy as jnp
from jax import lax
from jax.experimental import pallas as pl
from jax.experimental.pallas import tpu as pltpu
from jax.experimental.pallas import tpu_sc as plsc

N = 1_600_000
C = 50_000

NB = N // 128
BPC = 20
PB = BPC * 128
NCHUNKS = N // PB
NW = 32
TMAX = (NCHUNKS + NW - 1) // NW

CPAD = 50_176
G = CPAD // 128
GRID = G // 8
RPT = CPAD // 16
ZR = RPT // 4

_MESH = plsc.VectorSubcoreMesh(core_axis_name="c", subcore_axis_name="s")
_SC_PARAMS = pltpu.CompilerParams(use_tc_tiling_on_sc=False,
                                  needs_layout_passes=False)


def _zero_rows(ref, nrows):
    zero16 = jnp.zeros((16,), jnp.float32)

    def body(i, carry):
        ref[i, :] = zero16
        return carry

    lax.fori_loop(0, nrows, body, 0)


def _zero_rows8(ref, nrows):
    zero16 = jnp.zeros((16,), jnp.float32)
    iota = lax.iota(jnp.int32, 16)

    def body(k, carry):
        flat = k * 16 + iota
        plsc.store_scatter(ref, [lax.shift_right_logical(flat, 3),
                                 lax.bitwise_and(flat, 7)], zero16)
        return carry

    lax.fori_loop(0, nrows // 2, body, 0)


def _sc_prologue(acc, rowbuf, zbuf, sid):
    _zero_rows(rowbuf, 128)
    _zero_rows(zbuf, ZR)
    base = sid * RPT
    for r in range(4):
        pltpu.sync_copy(zbuf, acc.at[pl.ds(base + r * ZR, ZR), :])


def _fire_inputs(d3_hbm, seg_hbm, xbuf, ybuf, zbuf, sbuf,
                 slot, chunk, insem):
    blk0 = chunk * BPC
    pltpu.async_copy(d3_hbm.at[pl.ds(blk0, BPC), 1, :], xbuf.at[slot], insem)
    pltpu.async_copy(d3_hbm.at[pl.ds(blk0, BPC), 2, :], ybuf.at[slot], insem)
    pltpu.async_copy(d3_hbm.at[pl.ds(blk0, BPC), 3, :], zbuf.at[slot], insem)
    pltpu.async_copy(seg_hbm.at[pl.ds(blk0, BPC), :], sbuf.at[slot], insem)


def _drain_inputs(d3_hbm, seg_hbm, xbuf, ybuf, zbuf, sbuf,
                  slot, insem):
    pltpu.make_async_copy(d3_hbm.at[pl.ds(0, BPC), 1, :], xbuf.at[slot], insem).wait()
    pltpu.make_async_copy(d3_hbm.at[pl.ds(0, BPC), 2, :], ybuf.at[slot], insem).wait()
    pltpu.make_async_copy(d3_hbm.at[pl.ds(0, BPC), 3, :], zbuf.at[slot], insem).wait()
    pltpu.make_async_copy(seg_hbm.at[pl.ds(0, BPC), :], sbuf.at[slot], insem).wait()


@functools.partial(
    pl.kernel,
    out_type=jax.ShapeDtypeStruct((2, 10, CPAD), jnp.float32),
    mesh=_MESH,
    scratch_types=[
        pltpu.VMEM_SHARED((CPAD, 16), jnp.float32),
        pltpu.VMEM((2, BPC, 128), jnp.float32),
        pltpu.VMEM((2, BPC, 128), jnp.float32),
        pltpu.VMEM((2, BPC, 128), jnp.float32),
        pltpu.VMEM((2, BPC, 128), jnp.int32),
        pltpu.VMEM((2, 128, 16), jnp.float32),
        pltpu.VMEM((ZR, 16), jnp.float32),
        pltpu.VMEM((10, ZR), jnp.float32),
        pltpu.SemaphoreType.DMA,
        pltpu.SemaphoreType.DMA,
        pltpu.SemaphoreType.DMA,
    ],
    compiler_params=_SC_PARAMS,
)
def _sc_moments(d3_hbm, seg_hbm, mom_out,
                acc, xbuf, ybuf, zbuf, sbuf, rowbuf, zrow, trows,
                insem, scsem0, scsem1):
    cid = lax.axis_index("c")
    sid = lax.axis_index("s")
    wid = cid * 16 + sid
    _zero_rows(rowbuf.at[0], 128)
    _zero_rows(rowbuf.at[1], 128)
    _zero_rows(zrow, ZR)
    base0 = sid * RPT
    for r in range(4):
        pltpu.sync_copy(zrow, acc.at[pl.ds(base0 + r * ZR, ZR), :])
    plsc.subcore_barrier()
    iota = lax.iota(jnp.int32, 16)
    ones = jnp.full((16,), 1.0, jnp.float32)
    for r in range(2):
        for g in range(8):
            plsc.store_scatter(rowbuf.at[r],
                               [iota + g * 16, jnp.zeros((16,), jnp.int32)],
                               ones)
    scsems = (scsem0, scsem1)
    drain_dst = (rowbuf.at[0], rowbuf.at[1])

    _fire_inputs(d3_hbm, seg_hbm, xbuf, ybuf, zbuf, sbuf,
                 0, wid, insem)

    def chunk_body(t, carry):
        chunk = wid + NW * t
        slot = lax.rem(t, 2)

        @pl.when(chunk < NCHUNKS)
        def _():
            _drain_inputs(d3_hbm, seg_hbm, xbuf, ybuf, zbuf,
                          sbuf, slot, insem)
            nxt = chunk + NW

            @pl.when(nxt < NCHUNKS)
            def _():
                _fire_inputs(d3_hbm, seg_hbm, xbuf, ybuf, zbuf,
                             sbuf, 1 - slot, nxt, insem)

            xb = xbuf.at[slot]
            yb = ybuf.at[slot]
            zb = zbuf.at[slot]
            sb = sbuf.at[slot]

            def pair_body(p, c2):
                for r in range(2):
                    b = 2 * p + r
                    rb = rowbuf.at[r]

                    @pl.when((p > 0) | (t > 0))
                    def _():
                        pltpu.make_async_copy(
                            d3_hbm.at[pl.ds(0, 128), 0, pl.ds(0, 16)], rb,
                            scsems[r]).wait()
                    for g in range(8):
                        sl = pl.ds(g * 16, 16)
                        rr = iota + g * 16
                        vx = xb[b, sl]
                        vy = yb[b, sl]
                        vz = zb[b, sl]

                        def put(col, val):
                            plsc.store_scatter(
                                rb, [rr, jnp.full((16,), col, jnp.int32)], val)

                        put(1, vx)
                        put(2, vy)
                        put(3, vz)
                        put(4, vx * vx)
                        put(5, vy * vy)
                        put(6, vz * vz)
                        put(7, vx * vy)
                        put(8, vx * vz)
                        put(9, vy * vz)
                    pltpu.async_copy(rb, acc.at[sb.at[b]], scsems[r], add=True)
                return c2

            lax.fori_loop(0, BPC // 2, pair_body, 0)

        return carry

    lax.fori_loop(0, TMAX, chunk_body, 0)
    for r in range(2):
        pltpu.make_async_copy(d3_hbm.at[pl.ds(0, 128), 0, pl.ds(0, 16)],
                              drain_dst[r], scsems[r]).wait()
    plsc.subcore_barrier()
    for ch in range(4):
        base = sid * RPT + ch * ZR
        pltpu.sync_copy(acc.at[pl.ds(base, ZR), :], zrow)

        def grp_body(g2, c3):
            rows = iota + g2 * 16
            for j in range(10):
                v = plsc.load_gather(zrow, [rows, jnp.full((16,), j, jnp.int32)])
                trows[j, pl.ds(g2 * 16, 16)] = v
            return c3

        lax.fori_loop(0, ZR // 16, grp_body, 0)
        pltpu.sync_copy(trows, mom_out.at[cid, :, pl.ds(base, ZR)])


@functools.partial(
    pl.kernel,
    out_type=jax.ShapeDtypeStruct((2, CPAD), jnp.float32),
    mesh=_MESH,
    scratch_types=[
        pltpu.VMEM_SHARED((CPAD, 8), jnp.float32),
        pltpu.VMEM_SHARED((CPAD, 8), jnp.float32),
        pltpu.VMEM((2, BPC, 128), jnp.float32),
        pltpu.VMEM((2, BPC, 128), jnp.float32),
        pltpu.VMEM((2, BPC, 128), jnp.float32),
        pltpu.VMEM((2, BPC, 128), jnp.int32),
        pltpu.VMEM((2, 128, 8), jnp.float32),
        pltpu.VMEM((ZR, 8), jnp.float32),
        pltpu.VMEM((2, 128, 8), jnp.float32),
        pltpu.VMEM((ZR,), jnp.float32),
        pltpu.VMEM((6, 13, 128), jnp.float32),
        pltpu.VMEM((1664, 8), jnp.float32),
        pltpu.SemaphoreType.DMA,
        pltpu.SemaphoreType.DMA,
        pltpu.SemaphoreType.DMA,
        pltpu.SemaphoreType.DMA,
        pltpu.SemaphoreType.DMA,
    ],
    compiler_params=_SC_PARAMS,
)
def _sc_orient(d3_hbm, seg_hbm, params_hbm, sc_out,
               acc, ptab, xbuf, ybuf, zbuf, sbuf, rowbuf, zrow, prow, srow,
               pstage, pbuf,
               insem, scsem0, scsem1, gsem0, gsem1):
    cid = lax.axis_index("c")
    sid = lax.axis_index("s")
    wid = cid * 16 + sid
    iota = lax.iota(jnp.int32, 16)
    _zero_rows8(rowbuf.at[0], 128)
    _zero_rows8(rowbuf.at[1], 128)
    _zero_rows8(zrow, ZR)
    base0 = sid * RPT
    for r in range(4):
        pltpu.sync_copy(zrow, acc.at[pl.ds(base0 + r * ZR, ZR), :])

    def stage(gr0, nr):
        for j in range(6):
            pltpu.sync_copy(params_hbm.at[j, pl.ds(gr0, nr), :],
                            pstage.at[j, pl.ds(0, nr), :])

        def gg_body(gg, c0):
            row = lax.div(gg, jnp.int32(8))
            off = lax.rem(gg, jnp.int32(8)) * 16
            rr = iota + gg * 16
            for j in range(6):
                v = pstage[j, row, pl.ds(off, 16)]
                plsc.store_scatter(pbuf, [rr, jnp.full((16,), j, jnp.int32)], v)
            return c0

        lax.fori_loop(0, nr * 8, gg_body, 0)
        pltpu.sync_copy(pbuf.at[pl.ds(0, nr * 128), :],
                        ptab.at[pl.ds(gr0 * 128, nr * 128), :])

    @pl.when(sid < 8)
    def _():
        stage(sid * 25, 13)
        stage(sid * 25 + 13, 12)

    @pl.when(sid >= 8)
    def _():
        stage(200 + (sid - 8) * 24, 12)
        stage(200 + (sid - 8) * 24 + 12, 12)

    plsc.subcore_barrier()
    col0 = jnp.zeros((16,), jnp.int32)
    magic = jnp.full((16,), 0x5F3759DF, jnp.int32)
    one_i = jnp.full((16,), 1, jnp.int32)
    scsems = (scsem0, scsem1)
    gsems = (gsem0, gsem1)

    _fire_inputs(d3_hbm, seg_hbm, xbuf, ybuf, zbuf, sbuf,
                 0, wid, insem)

    def chunk_body(t, carry):
        chunk = wid + NW * t
        slot = lax.rem(t, 2)

        @pl.when(chunk < NCHUNKS)
        def _():
            _drain_inputs(d3_hbm, seg_hbm, xbuf, ybuf, zbuf,
                          sbuf, slot, insem)
            nxt = chunk + NW

            @pl.when(nxt < NCHUNKS)
            def _():
                _fire_inputs(d3_hbm, seg_hbm, xbuf, ybuf, zbuf,
                             sbuf, 1 - slot, nxt, insem)

            xb = xbuf.at[slot]
            yb = ybuf.at[slot]
            zb = zbuf.at[slot]
            sb = sbuf.at[slot]
            pltpu.async_copy(ptab.at[sb.at[0]], prow.at[0], gsem0)

            def pair_body(p, c2):
                for r in range(2):
                    b = 2 * p + r
                    rb = rowbuf.at[r]
                    pb = prow.at[r]
                    pltpu.make_async_copy(
                        d3_hbm.at[pl.ds(0, 128), 0, pl.ds(0, 8)], pb,
                        gsems[r]).wait()

                    @pl.when(b + 1 < BPC)
                    def _():
                        pltpu.async_copy(ptab.at[sb.at[b + 1]],
                                         prow.at[1 - r], gsems[1 - r])

                    @pl.when((p > 0) | (t > 0))
                    def _():
                        pltpu.make_async_copy(
                            d3_hbm.at[pl.ds(0, 128), 0, pl.ds(0, 8)], rb,
                            scsems[r]).wait()
                    for g in range(8):
                        sl = pl.ds(g * 16, 16)
                        rr = iota + g * 16
                        x = xb[b, sl]
                        y = yb[b, sl]
                        z = zb[b, sl]
                        cx = plsc.load_gather(pb, [rr, jnp.full((16,), 0, jnp.int32)])
                        cy = plsc.load_gather(pb, [rr, jnp.full((16,), 1, jnp.int32)])
                        cz = plsc.load_gather(pb, [rr, jnp.full((16,), 2, jnp.int32)])
                        vx = plsc.load_gather(pb, [rr, jnp.full((16,), 3, jnp.int32)])
                        vy = plsc.load_gather(pb, [rr, jnp.full((16,), 4, jnp.int32)])
                        vz = plsc.load_gather(pb, [rr, jnp.full((16,), 5, jnp.int32)])
                        xcx = x - cx
                        xcy = y - cy
                        xcz = z - cz
                        x0 = xcx * vx + xcy * vy + xcz * vz
                        d = xcx * xcx + xcy * xcy + xcz * xcz - x0 * x0
                        d = jnp.maximum(d, 0.0)
                        rv = plsc.bitcast(magic - lax.shift_right_logical(
                            plsc.bitcast(d, jnp.int32), one_i), jnp.float32)
                        for _ in range(3):
                            h = 0.5 * d * rv
                            rv = rv * (1.5 - h * rv)
                        t_val = x0 * (d * rv)
                        plsc.store_scatter(rb, [rr, col0], t_val)
                    pltpu.async_copy(rb, acc.at[sb.at[b]], scsems[r], add=True)
                return c2

            lax.fori_loop(0, BPC // 2, pair_body, 0)

        return carry

    lax.fori_loop(0, TMAX, chunk_body, 0)
    for r in range(2):
        pltpu.make_async_copy(d3_hbm.at[pl.ds(0, 128), 0, pl.ds(0, 8)],
                              rowbuf.at[r], scsems[r]).wait()
    plsc.subcore_barrier()
    for ch in range(4):
        base = sid * RPT + ch * ZR
        pltpu.sync_copy(acc.at[pl.ds(base, ZR), :], zrow)

        def grp_body(g2, c3):
            v = plsc.load_gather(zrow, [iota + g2 * 16, col0])
            srow[pl.ds(g2 * 16, 16)] = v
            return c3

        lax.fori_loop(0, ZR // 16, grp_body, 0)
        pltpu.sync_copy(srow, sc_out.at[cid, pl.ds(base, ZR)])


def _jacobi_rot(app, aqq, apq):
    small = jnp.abs(apq) <= 1e-30
    apq_s = jnp.where(small, 1.0, apq)
    tau = (aqq - app) / (2.0 * apq_s)
    t = jnp.sign(tau) / (jnp.abs(tau) + jnp.sqrt(1.0 + tau * tau))
    t = jnp.where(tau == 0.0, 1.0, t)
    c = 1.0 / jnp.sqrt(1.0 + t * t)
    s = t * c
    c = jnp.where(small, 1.0, c)
    s = jnp.where(small, 0.0, s)
    return c, s


def _tc1_body(momref, featref, parref):
    m = [momref[0, j] + momref[1, j] for j in range(10)]
    n = m[0]
    n_safe = jnp.maximum(n, 1.0)
    sx, sy, sz = m[1], m[2], m[3]
    cx, cy, cz = sx / n_safe, sy / n_safe, sz / n_safe
    a00 = m[4] - sx * cx
    a11 = m[5] - sy * cy
    a22 = m[6] - sz * cz
    a01 = m[7] - sx * cy
    a02 = m[8] - sx * cz
    a12 = m[9] - sy * cz
    safe = n >= 2.0
    a00 = jnp.where(safe, a00, 1.0)
    a11 = jnp.where(safe, a11, 2.0)
    a22 = jnp.where(safe, a22, 3.0)
    a01 = jnp.where(safe, a01, 0.0)
    a02 = jnp.where(safe, a02, 0.0)
    a12 = jnp.where(safe, a12, 0.0)
    g00, g01, g02, g11, g12, g22 = a00, a01, a02, a11, a12, a22

    one = jnp.ones_like(a00)
    zero = jnp.zeros_like(a00)
    v00, v01, v02 = one, zero, zero
    v10, v11, v12 = zero, one, zero
    v20, v21, v22 = zero, zero, one

    for _ in range(3):
        c, s = _jacobi_rot(a00, a11, a01)
        a00, a11 = (c * c * a00 - 2 * s * c * a01 + s * s * a11,
                    s * s * a00 + 2 * s * c * a01 + c * c * a11)
        a02, a12 = c * a02 - s * a12, s * a02 + c * a12
        a01 = zero
        v00, v01 = c * v00 - s * v01, s * v00 + c * v01
        v10, v11 = c * v10 - s * v11, s * v10 + c * v11
        v20, v21 = c * v20 - s * v21, s * v20 + c * v21

        c, s = _jacobi_rot(a00, a22, a02)
        a00, a22 = (c * c * a00 - 2 * s * c * a02 + s * s * a22,
                    s * s * a00 + 2 * s * c * a02 + c * c * a22)
        a01, a12 = c * a01 - s * a12, s * a01 + c * a12
        a02 = zero
        v00, v02 = c * v00 - s * v02, s * v00 + c * v02
        v10, v12 = c * v10 - s * v12, s * v10 + c * v12
        v20, v22 = c * v20 - s * v22, s * v20 + c * v22

        c, s = _jacobi_rot(a11, a22, a12)
        a11, a22 = (c * c * a11 - 2 * s * c * a12 + s * s * a22,
                    s * s * a11 + 2 * s * c * a12 + c * c * a22)
        a01, a02 = c * a01 - s * a02, s * a01 + c * a02
        a12 = zero
        v01, v02 = c * v01 - s * v02, s * v01 + c * v02
        v11, v12 = c * v11 - s * v12, s * v11 + c * v12
        v21, v22 = c * v21 - s * v22, s * v21 + c * v22

    d0, d1, d2 = a00, a11, a22
    w2 = jnp.maximum(jnp.maximum(d0, d1), d2)
    w0 = jnp.minimum(jnp.minimum(d0, d1), d2)
    w1 = d0 + d1 + d2 - w2 - w0
    is0 = (d0 >= d1) & (d0 >= d2)
    is1 = jnp.logical_not(is0) & (d1 >= d2)
    v0x = jnp.where(is0, v00, jnp.where(is1, v01, v02))
    v0y = jnp.where(is0, v10, jnp.where(is1, v11, v12))
    v0z = jnp.where(is0, v20, jnp.where(is1, v21, v22))

    w2s = jnp.where(w2 != 0.0, w2, 1.0)
    dirwt = 1.0 - w1 / w2s

    feats = [
        jnp.where(safe, cx, sx),
        jnp.where(safe, cy, sy),
        jnp.where(safe, cz, sz),
        jnp.where(safe, g00 / w2s, 0.0),
        jnp.where(safe, g01 / w2s, 0.0),
        jnp.where(safe, g02 / w2s, 0.0),
        jnp.where(safe, g01 / w2s, 0.0),
        jnp.where(safe, g11 / w2s, 0.0),
        jnp.where(safe, g12 / w2s, 0.0),
        jnp.where(safe, g02 / w2s, 0.0),
        jnp.where(safe, g12 / w2s, 0.0),
        jnp.where(safe, g22 / w2s, 0.0),
        jnp.where(safe, dirwt * v0x, 0.0),
        jnp.where(safe, dirwt * v0y, 0.0),
        jnp.where(safe, dirwt * v0z, 0.0),
        n,
    ]
    for j in range(16):
        featref[j] = feats[j]
    pars = [cx, cy, cz, v0x, v0y, v0z]
    for j in range(6):
        parref[j] = pars[j]


def _tc1(momT):
    return pl.pallas_call(
        _tc1_body,
        grid=(7,),
        in_specs=[pl.BlockSpec((2, 10, 56, 128), lambda i: (0, 0, i, 0))],
        out_specs=[pl.BlockSpec((16, 56, 128), lambda i: (0, i, 0)),
                   pl.BlockSpec((6, 56, 128), lambda i: (0, i, 0))],
        out_shape=[jax.ShapeDtypeStruct((16, G, 128), jnp.float32),
                   jax.ShapeDtypeStruct((6, G, 128), jnp.float32)],
    )(momT)


def _tc2_body(featref, scref, outref):
    sc = scref[0] + scref[1]
    n = featref[3]
    flip = (n >= 2.0) & (sc < 0.0)
    fac = jnp.where(flip, -1.0, 1.0)
    outref[0] = featref[0] * fac
    outref[1] = featref[1] * fac
    outref[2] = featref[2] * fac
    outref[3] = n


def _tc2(feats0T, scs):
    return pl.pallas_call(
        _tc2_body,
        grid=(7,),
        in_specs=[pl.BlockSpec((4, 56, 128), lambda i: (3, i, 0)),
                  pl.BlockSpec((2, 56, 128), lambda i: (0, i, 0))],
        out_specs=pl.BlockSpec((4, 56, 128), lambda i: (3, i, 0)),
        out_shape=jax.ShapeDtypeStruct((16, G, 128), jnp.float32),
        input_output_aliases={0: 0},
    )(feats0T, scs)


def kernel(data, segment_ids):
    seg2d = segment_ids.astype(jnp.int32).reshape(NB, 128)
    d3 = jnp.pad(data, ((0, 0), (0, 3))).reshape(NB, 128, 8).transpose(0, 2, 1)

    mom = _sc_moments(d3, seg2d)
    momT = mom.reshape(2, 10, G, 128)
    feats0T, params = _tc1(momT)

    sc_acc = _sc_orient(d3, seg2d, params)
    scs = sc_acc.reshape(2, G, 128)
    outT = _tc2(feats0T, scs)
    return outT.reshape(16, CPAD)[:, :C].T
